# Initial kernel scaffold; baseline (speedup 1.0000x reference)
import jax
import jax.numpy as jnp
from jax.experimental import pallas as pl


def kernel(x_context, edge_index_context, center_mask_context, x_target, edge_index_target, center_mask_target, cW1, cb1, cW2, cb2, tW1, tb1, tW2, tb2, pW1, pb1, pW2, pb2):
    raise NotImplementedError("write your pallas kernel here")


# speedup vs baseline: 4.0649x; 4.0649x over previous
import jax
import jax.numpy as jnp
from jax import lax
from jax.experimental import pallas as pl
from jax.experimental.pallas import tpu as pltpu
from jax.experimental.pallas import tpu_sc as plsc

NC = 2
NS = 16
NW = NC * NS

_SC_PARAMS = pltpu.CompilerParams(needs_layout_passes=False)


def _seg_sum_kernel(n_pad, feat, n_edges, with_deg):
  epw = n_edges // NW
  K = 80
  nchunk = epw // K
  assert epw * NW == n_edges and K * nchunk == epw
  assert n_pad % (NS * 128) == 0
  rps = n_pad // NS
  mesh = plsc.VectorSubcoreMesh(core_axis_name="c", subcore_axis_name="s")
  out_type = [jax.ShapeDtypeStruct((n_pad, feat), jnp.float32),
              jax.ShapeDtypeStruct((n_pad, feat), jnp.float32)]
  if with_deg:
    out_type.append(jax.ShapeDtypeStruct((NW, n_pad), jnp.float32))
  scratch = [
      pltpu.VMEM((1, K), jnp.int32),
      pltpu.VMEM((1, K), jnp.int32),
      pltpu.VMEM((1, K, feat), jnp.float32),
      pltpu.VMEM((64, feat), jnp.float32),
      pltpu.VMEM_SHARED((n_pad, feat), jnp.float32),
      pltpu.SemaphoreType.DMA,
  ]
  if with_deg:
    scratch.append(pltpu.VMEM((n_pad,), jnp.float32))

  def body(table, src, dst, acc0, acc1, *rest):
    if with_deg:
      degp = rest[0]
      rest = rest[1:]
    srcb, dstb, rows, zbuf, acc_sh, gsem = rest[:6]
    degbuf = rest[6] if with_deg else None
    c = lax.axis_index("c")
    s = lax.axis_index("s")
    wid = c * NS + s
    base = wid * epw
    z16 = jnp.zeros((16,), jnp.float32)
    gpr = feat // 16

    def zb(t, carry):
      zbuf[t // gpr, pl.ds((t % gpr) * 16, 16)] = z16
      return carry
    lax.fori_loop(0, 64 * gpr, zb, 0)
    for r in range(rps // 64):
      pltpu.sync_copy(zbuf, acc_sh.at[pl.ds(s * rps + r * 64, 64)])
    if with_deg:
      def zd(t, carry):
        degbuf[pl.ds(t * 16, 16)] = z16
        return carry
      lax.fori_loop(0, n_pad // 16, zd, 0)
    plsc.subcore_barrier()

    ones = jnp.ones((16,), jnp.float32)

    def chunk(j, carry):
      off = pl.multiple_of(base + j * K, 8)
      pltpu.sync_copy(src.at[pl.ds(off, K)], srcb.at[0])
      pltpu.sync_copy(dst.at[pl.ds(off, K)], dstb.at[0])
      pltpu.async_copy(table.at[srcb.at[0]], rows.at[0], gsem).wait()
      pltpu.sync_copy(rows.at[0], acc_sh.at[dstb.at[0]], add=True)
      if with_deg:
        for g in range(K // 16):
          idx = dstb.at[0][pl.ds(g * 16, 16)]
          plsc.addupdate_scatter(degbuf, [idx], ones)
      return carry
    lax.fori_loop(0, nchunk, chunk, 0)
    plsc.subcore_barrier()

    @pl.when(c == 0)
    def _():
      for r in range(rps // 128):
        sl = pl.ds(s * rps + r * 128, 128)
        pltpu.sync_copy(acc_sh.at[sl], acc0.at[sl])

    @pl.when(c == 1)
    def _():
      for r in range(rps // 128):
        sl = pl.ds(s * rps + r * 128, 128)
        pltpu.sync_copy(acc_sh.at[sl], acc1.at[sl])

    if with_deg:
      pltpu.sync_copy(degbuf, degp.at[wid])

  return pl.kernel(body, out_type=tuple(out_type), mesh=mesh,
                   scratch_types=tuple(scratch), compiler_params=_SC_PARAMS)


def _center_gather_kernel(n_pad, feat, n_b):
  bpw = n_b // NW
  assert bpw * NW == n_b and bpw % 16 == 0
  mesh = plsc.VectorSubcoreMesh(core_axis_name="c", subcore_axis_name="s")
  out_type = (jax.ShapeDtypeStruct((n_b, feat), jnp.float32),
              jax.ShapeDtypeStruct((n_b, feat), jnp.float32),
              jax.ShapeDtypeStruct((n_b,), jnp.float32))
  scratch = (
      pltpu.VMEM((1, bpw), jnp.int32),
      pltpu.VMEM((bpw, feat), jnp.float32),
      pltpu.VMEM((bpw, feat), jnp.float32),
      pltpu.VMEM((n_pad,), jnp.float32),
      pltpu.VMEM((bpw,), jnp.float32),
      pltpu.SemaphoreType.DMA,
      pltpu.SemaphoreType.DMA,
  )

  def body(a0, a1, invdeg, center, r0, r1, idegc,
           idxb, rows0, rows1, degv, degc, s0, s1):
    c = lax.axis_index("c")
    s = lax.axis_index("s")
    base = (c * NS + s) * bpw
    pltpu.sync_copy(center.at[pl.ds(base, bpw)], idxb.at[0])
    cp0 = pltpu.async_copy(a0.at[idxb.at[0]], rows0, s0)
    cp1 = pltpu.async_copy(a1.at[idxb.at[0]], rows1, s1)
    pltpu.sync_copy(invdeg, degv)
    for g in range(bpw // 16):
      iv = idxb.at[0][pl.ds(g * 16, 16)]
      degc[pl.ds(g * 16, 16)] = plsc.load_gather(degv, [iv])
    cp0.wait()
    cp1.wait()
    pltpu.sync_copy(rows0, r0.at[pl.ds(base, bpw)])
    pltpu.sync_copy(rows1, r1.at[pl.ds(base, bpw)])
    pltpu.sync_copy(degc, idegc.at[pl.ds(base, bpw)])

  return pl.kernel(body, out_type=out_type, mesh=mesh, scratch_types=scratch,
                   compiler_params=_SC_PARAMS)


def _conv_dense(n_pad, d_in, d_out):
  blk = 1280
  grid = (n_pad // blk,)

  def body(a0, a1, degp, w, bb, h, invd):
    deg = jnp.sum(degp[...], axis=1)
    inv = 1.0 / jnp.maximum(deg, 1.0)
    agg = (a0[...] + a1[...]) * inv[:, None]
    h[...] = jnp.maximum(
        jnp.dot(agg, w[...], preferred_element_type=jnp.float32) + bb[...], 0.0)
    invd[...] = inv[:, None]

  return pl.pallas_call(
      body,
      grid=grid,
      in_specs=[
          pl.BlockSpec((blk, d_in), lambda i: (i, 0)),
          pl.BlockSpec((blk, d_in), lambda i: (i, 0)),
          pl.BlockSpec((blk, NW), lambda i: (i, 0)),
          pl.BlockSpec((d_in, d_out), lambda i: (0, 0)),
          pl.BlockSpec((1, d_out), lambda i: (0, 0)),
      ],
      out_specs=[
          pl.BlockSpec((blk, d_out), lambda i: (i, 0)),
          pl.BlockSpec((blk, 1), lambda i: (i, 0)),
      ],
      out_shape=[
          jax.ShapeDtypeStruct((n_pad, d_out), jnp.float32),
          jax.ShapeDtypeStruct((n_pad, 1), jnp.float32),
      ],
  )


def _final_kernel(h, n_b, h2):
  def body(r0c, r1c, idc, r0t, r1t, idt, cw2, cb2, tw2, tb2,
           pw1, pb1, pw2, pb2, zp, zt):
    zc = jnp.dot((r0c[...] + r1c[...]) * idc[...], cw2[...],
                 preferred_element_type=jnp.float32) + cb2[...]
    hid = jnp.maximum(
        jnp.dot(zc, pw1[...], preferred_element_type=jnp.float32) + pb1[...],
        0.0)
    zp[...] = jnp.dot(hid, pw2[...],
                      preferred_element_type=jnp.float32) + pb2[...]
    zt[...] = jnp.dot((r0t[...] + r1t[...]) * idt[...], tw2[...],
                      preferred_element_type=jnp.float32) + tb2[...]

  return pl.pallas_call(
      body,
      out_shape=[jax.ShapeDtypeStruct((n_b, h), jnp.float32),
                 jax.ShapeDtypeStruct((n_b, h), jnp.float32)])


def kernel(x_context, edge_index_context, center_mask_context,
           x_target, edge_index_target, center_mask_target,
           cW1, cb1, cW2, cb2, tW1, tb1, tW2, tb2,
           pW1, pb1, pW2, pb2):
  n, d = x_context.shape
  e = edge_index_context.shape[1]
  nb = center_mask_context.shape[0]
  h = cW1.shape[1]
  h2 = pW1.shape[1]
  n_pad = -(-n // (NS * 128)) * (NS * 128)

  seg_deg = _seg_sum_kernel(n_pad, d, e, True)
  seg = _seg_sum_kernel(n_pad, h, e, False)
  conv = _conv_dense(n_pad, d, h)
  gath = _center_gather_kernel(n_pad, h, nb)
  fin = _final_kernel(h, nb, h2)

  def enc(x, ei, ctr, w1, bias1):
    src, dst = ei[0], ei[1]
    a0, a1, degp = seg_deg(x, src, dst)
    hdn, invd = conv(a0, a1, degp.T, w1, bias1.reshape(1, h))
    b0, b1 = seg(hdn, src, dst)
    r0, r1, idc = gath(b0, b1, invd.reshape(-1), ctr)
    return r0, r1, idc.reshape(nb, 1)

  r0c, r1c, idc = enc(x_context, edge_index_context, center_mask_context,
                      cW1, cb1)
  r0t, r1t, idt = enc(x_target, edge_index_target, center_mask_target,
                      tW1, tb1)
  zp, zt = fin(r0c, r1c, idc, r0t, r1t, idt,
               cW2, cb2.reshape(1, h), tW2, tb2.reshape(1, h),
               pW1, pb1.reshape(1, h2), pW2, pb2.reshape(1, h))
  return (zp, zt)


# speedup vs baseline: 10.3667x; 2.5503x over previous
import jax
import jax.numpy as jnp
from jax import lax
from jax.experimental import pallas as pl
from jax.experimental.pallas import tpu as pltpu
from jax.experimental.pallas import tpu_sc as plsc

NC = 2
NS = 16
NW = NC * NS

_SC_PARAMS = pltpu.CompilerParams(needs_layout_passes=False)


_K = 40
_G = 5


def _seg_sum_kernel(n_pad, feat, n_edges, with_deg):
  K, G = _K, _G
  epw = n_edges // NW
  nchunk = epw // K
  ng = nchunk // G
  nouter = ng // 2
  assert epw * NW == n_edges and K * nchunk == epw and G * ng == nchunk
  assert ng % 2 == 0 and n_pad % (NS * 128) == 0
  rps = n_pad // NS
  mesh = plsc.VectorSubcoreMesh(core_axis_name="c", subcore_axis_name="s")
  out_type = [jax.ShapeDtypeStruct((n_pad, feat), jnp.float32),
              jax.ShapeDtypeStruct((n_pad, feat), jnp.float32)]
  if with_deg:
    out_type.append(jax.ShapeDtypeStruct((NW, n_pad), jnp.float32))
  scratch = [pltpu.VMEM((1, K), jnp.int32)] * (2 * G) \
    + [pltpu.VMEM((1, K), jnp.int32)] * (2 * G) \
    + [
      pltpu.VMEM((G, K, feat), jnp.float32),
      pltpu.VMEM((64, feat), jnp.float32),
      pltpu.VMEM_SHARED((n_pad, feat), jnp.float32),
      pltpu.SemaphoreType.DMA,
      pltpu.SemaphoreType.DMA,
  ] + [pltpu.SemaphoreType.DMA] * G \
    + [pltpu.SemaphoreType.DMA] * G \
    + [pltpu.SemaphoreType.DMA]
  if with_deg:
    scratch.append(pltpu.VMEM((n_pad,), jnp.float32))

  def body(table, src_r, dst_r, acc0, acc1, *rest):
    if with_deg:
      degp = rest[0]
      rest = rest[1:]
    srcb = rest[:2 * G]
    dstb = rest[2 * G:4 * G]
    rows, zbuf, acc_sh = rest[4 * G:4 * G + 3]
    off0 = 4 * G + 3
    isem = rest[off0:off0 + 2]
    gsem = rest[off0 + 2:off0 + 2 + G]
    ssem = rest[off0 + 2 + G:off0 + 2 + 2 * G]
    zsem = rest[off0 + 2 + 2 * G]
    degbuf = rest[off0 + 3 + 2 * G] if with_deg else None
    c = lax.axis_index("c")
    s = lax.axis_index("s")
    wid = c * NS + s
    wbase = wid * epw
    z16 = jnp.zeros((16,), jnp.float32)
    ones = jnp.ones((16,), jnp.float32)
    tailmask = lax.broadcasted_iota(jnp.int32, (16,), 0) >= 8
    gpr = feat // 16

    def zb(t, carry):
      zbuf[t // gpr, pl.ds((t % gpr) * 16, 16)] = z16
      return carry
    lax.fori_loop(0, 64 * gpr, zb, 0)
    for r in range(rps // 64):
      pltpu.make_async_copy(
          zbuf, acc_sh.at[pl.ds(s * rps + r * 64, 64)], zsem).start()
    if with_deg:
      def zd(t, carry):
        degbuf[pl.ds(t * 16, 16)] = z16
        return carry
      lax.fori_loop(0, n_pad // 16, zd, 0)
    for r in range(rps // 64):
      pltpu.make_async_copy(
          zbuf, acc_sh.at[pl.ds(s * rps + r * 64, 64)], zsem).wait()
    plsc.subcore_barrier()

    def fire_idx_group(gi, q):
      off = pl.multiple_of(wbase + gi * (G * K), 8)
      for uu in range(G):
        pltpu.make_async_copy(src_r.at[pl.ds(off + uu * K, K)],
                              srcb[q * G + uu].at[0], isem[q]).start()
        pltpu.make_async_copy(dst_r.at[pl.ds(off + uu * K, K)],
                              dstb[q * G + uu].at[0], isem[q]).start()

    def wait_idx_group(q):
      off = pl.multiple_of(wbase, 8)
      for uu in range(G):
        pltpu.make_async_copy(src_r.at[pl.ds(off + uu * K, K)],
                              srcb[q * G + uu].at[0], isem[q]).wait()
        pltpu.make_async_copy(dst_r.at[pl.ds(off + uu * K, K)],
                              dstb[q * G + uu].at[0], isem[q]).wait()

    def fire_gather(q, u):
      pltpu.make_async_copy(table.at[srcb[q * G + u].at[0]],
                            rows.at[u], gsem[u]).start()

    def wait_gather(u):
      pltpu.make_async_copy(table.at[srcb[u].at[0]],
                            rows.at[u], gsem[u]).wait()

    def fire_scatter(q, u):
      pltpu.async_copy(rows.at[u], acc_sh.at[dstb[q * G + u].at[0]],
                       ssem[u], add=True)

    def wait_scatter(u):
      pltpu.make_async_copy(rows.at[u], acc_sh.at[dstb[u].at[0]],
                            ssem[u]).wait()

    def drain_chunk(q, u):
      wait_gather(u)
      fire_scatter(q, u)
      if with_deg:
        d = dstb[q * G + u]
        plsc.addupdate_scatter(degbuf, [d[0, pl.ds(0, 16)]], ones)
        plsc.addupdate_scatter(degbuf, [d[0, pl.ds(16, 16)]], ones)
        plsc.addupdate_scatter(degbuf, [d[0, pl.ds(24, 16)]], ones,
                               mask=tailmask)

    offp = pl.multiple_of(wbase, 8)
    for uu in range(G):
      pltpu.sync_copy(src_r.at[pl.ds(offp + uu * K, K)], srcb[uu].at[0])
      pltpu.sync_copy(dst_r.at[pl.ds(offp + uu * K, K)], dstb[uu].at[0])
    fire_idx_group(1, 1)
    fire_gather(0, 0)
    fire_gather(0, 1)

    def outer(gg, carry):
      for pp in (0, 1):
        for u in range(G):
          if u < 3:
            w = u + 2
            if pp == 0:
              @pl.when(gg > 0)
              def _(w=w):
                wait_scatter(w)
            else:
              wait_scatter(w)
            fire_gather(pp, w)
          else:
            w = u - 3
            if pp == 0:
              if u == 3:
                wait_idx_group(1)
              wait_scatter(w)
              fire_gather(1, w)
            else:
              @pl.when(gg < nouter - 1)
              def _(w=w, u=u):
                if u == 3:
                  wait_idx_group(0)
                wait_scatter(w)
                fire_gather(0, w)
          drain_chunk(pp, u)
          if u == 4:
            @pl.when(gg < nouter - 1)
            def _(pp=pp):
              fire_idx_group(gg * 2 + 2 + pp, pp)
      return carry

    lax.fori_loop(0, nouter, outer, 0)
    for u in range(G):
      wait_scatter(u)
    plsc.subcore_barrier()

    @pl.when(c == 0)
    def _():
      for r in range(rps // 128):
        sl = pl.ds(s * rps + r * 128, 128)
        pltpu.make_async_copy(acc_sh.at[sl], acc0.at[sl], zsem).start()
      for r in range(rps // 128):
        sl = pl.ds(s * rps + r * 128, 128)
        pltpu.make_async_copy(acc_sh.at[sl], acc0.at[sl], zsem).wait()

    @pl.when(c == 1)
    def _():
      for r in range(rps // 128):
        sl = pl.ds(s * rps + r * 128, 128)
        pltpu.make_async_copy(acc_sh.at[sl], acc1.at[sl], zsem).start()
      for r in range(rps // 128):
        sl = pl.ds(s * rps + r * 128, 128)
        pltpu.make_async_copy(acc_sh.at[sl], acc1.at[sl], zsem).wait()

    if with_deg:
      pltpu.sync_copy(degbuf, degp.at[wid])

  return pl.kernel(body, out_type=tuple(out_type), mesh=mesh,
                   scratch_types=tuple(scratch), compiler_params=_SC_PARAMS)


def _center_gather_kernel(n_pad, feat, n_b):
  bpw = n_b // NW
  assert bpw * NW == n_b and bpw % 16 == 0
  mesh = plsc.VectorSubcoreMesh(core_axis_name="c", subcore_axis_name="s")
  out_type = (jax.ShapeDtypeStruct((n_b, feat), jnp.float32),
              jax.ShapeDtypeStruct((n_b, feat), jnp.float32),
              jax.ShapeDtypeStruct((n_b,), jnp.float32))
  scratch = (
      pltpu.VMEM((1, bpw), jnp.int32),
      pltpu.VMEM((bpw, feat), jnp.float32),
      pltpu.VMEM((bpw, feat), jnp.float32),
      pltpu.VMEM((n_pad,), jnp.float32),
      pltpu.VMEM((bpw,), jnp.float32),
      pltpu.SemaphoreType.DMA,
      pltpu.SemaphoreType.DMA,
  )

  def body(a0, a1, invdeg, center, r0, r1, idegc,
           idxb, rows0, rows1, degv, degc, s0, s1):
    c = lax.axis_index("c")
    s = lax.axis_index("s")
    base = (c * NS + s) * bpw
    pltpu.sync_copy(center.at[pl.ds(base, bpw)], idxb.at[0])
    cp0 = pltpu.async_copy(a0.at[idxb.at[0]], rows0, s0)
    cp1 = pltpu.async_copy(a1.at[idxb.at[0]], rows1, s1)
    pltpu.sync_copy(invdeg, degv)
    for g in range(bpw // 16):
      iv = idxb.at[0][pl.ds(g * 16, 16)]
      degc[pl.ds(g * 16, 16)] = plsc.load_gather(degv, [iv])
    cp0.wait()
    cp1.wait()
    pltpu.sync_copy(rows0, r0.at[pl.ds(base, bpw)])
    pltpu.sync_copy(rows1, r1.at[pl.ds(base, bpw)])
    pltpu.sync_copy(degc, idegc.at[pl.ds(base, bpw)])

  return pl.kernel(body, out_type=out_type, mesh=mesh, scratch_types=scratch,
                   compiler_params=_SC_PARAMS)


def _conv_dense(n_pad, d_in, d_out):
  blk = 1280
  grid = (n_pad // blk,)

  def body(a0, a1, degp, w, bb, h, invd):
    deg = jnp.sum(degp[...], axis=1)
    inv = 1.0 / jnp.maximum(deg, 1.0)
    agg = (a0[...] + a1[...]) * inv[:, None]
    h[...] = jnp.maximum(
        jnp.dot(agg, w[...], preferred_element_type=jnp.float32) + bb[...], 0.0)
    invd[...] = inv[:, None]

  return pl.pallas_call(
      body,
      grid=grid,
      in_specs=[
          pl.BlockSpec((blk, d_in), lambda i: (i, 0)),
          pl.BlockSpec((blk, d_in), lambda i: (i, 0)),
          pl.BlockSpec((blk, NW), lambda i: (i, 0)),
          pl.BlockSpec((d_in, d_out), lambda i: (0, 0)),
          pl.BlockSpec((1, d_out), lambda i: (0, 0)),
      ],
      out_specs=[
          pl.BlockSpec((blk, d_out), lambda i: (i, 0)),
          pl.BlockSpec((blk, 1), lambda i: (i, 0)),
      ],
      out_shape=[
          jax.ShapeDtypeStruct((n_pad, d_out), jnp.float32),
          jax.ShapeDtypeStruct((n_pad, 1), jnp.float32),
      ],
  )


def _final_kernel(h, n_b, h2):
  def body(r0c, r1c, idc, r0t, r1t, idt, cw2, cb2, tw2, tb2,
           pw1, pb1, pw2, pb2, zp, zt):
    zc = jnp.dot((r0c[...] + r1c[...]) * idc[...], cw2[...],
                 preferred_element_type=jnp.float32) + cb2[...]
    hid = jnp.maximum(
        jnp.dot(zc, pw1[...], preferred_element_type=jnp.float32) + pb1[...],
        0.0)
    zp[...] = jnp.dot(hid, pw2[...],
                      preferred_element_type=jnp.float32) + pb2[...]
    zt[...] = jnp.dot((r0t[...] + r1t[...]) * idt[...], tw2[...],
                      preferred_element_type=jnp.float32) + tb2[...]

  return pl.pallas_call(
      body,
      out_shape=[jax.ShapeDtypeStruct((n_b, h), jnp.float32),
                 jax.ShapeDtypeStruct((n_b, h), jnp.float32)])


def kernel(x_context, edge_index_context, center_mask_context,
           x_target, edge_index_target, center_mask_target,
           cW1, cb1, cW2, cb2, tW1, tb1, tW2, tb2,
           pW1, pb1, pW2, pb2):
  n, d = x_context.shape
  e = edge_index_context.shape[1]
  nb = center_mask_context.shape[0]
  h = cW1.shape[1]
  h2 = pW1.shape[1]
  n_pad = -(-n // (NS * 128)) * (NS * 128)

  seg_deg = _seg_sum_kernel(n_pad, d, e, True)
  seg = _seg_sum_kernel(n_pad, h, e, False)
  conv = _conv_dense(n_pad, d, h)
  gath = _center_gather_kernel(n_pad, h, nb)
  fin = _final_kernel(h, nb, h2)

  def enc(x, ei, ctr, w1, bias1):
    src, dst = ei[0], ei[1]
    a0, a1, degp = seg_deg(x, src, dst)
    hdn, invd = conv(a0, a1, degp.T, w1, bias1.reshape(1, h))
    b0, b1 = seg(hdn, src, dst)
    r0, r1, idc = gath(b0, b1, invd.reshape(-1), ctr)
    return r0, r1, idc.reshape(nb, 1)

  r0c, r1c, idc = enc(x_context, edge_index_context, center_mask_context,
                      cW1, cb1)
  r0t, r1t, idt = enc(x_target, edge_index_target, center_mask_target,
                      tW1, tb1)
  zp, zt = fin(r0c, r1c, idc, r0t, r1t, idt,
               cW2, cb2.reshape(1, h), tW2, tb2.reshape(1, h),
               pW1, pb1.reshape(1, h2), pW2, pb2.reshape(1, h))
  return (zp, zt)


# speedup vs baseline: 14.3737x; 1.3865x over previous
import jax
import jax.numpy as jnp
from jax import lax
from jax.experimental import pallas as pl
from jax.experimental.pallas import tpu as pltpu
from jax.experimental.pallas import tpu_sc as plsc

NC = 2
NS = 16
NW = NC * NS

_SC_PARAMS = pltpu.CompilerParams(needs_layout_passes=False)


_K = 40
_G = 5


def _seg_sum_kernel(n_pad, feat, n_edges, with_deg):
  K, G = _K, _G
  epw = n_edges // NW
  nchunk = epw // K
  ng = nchunk // G
  nouter = ng // 2
  assert epw * NW == n_edges and K * nchunk == epw and G * ng == nchunk
  assert ng % 2 == 0 and n_pad % (NS * 128) == 0
  rps = n_pad // NS
  mesh = plsc.VectorSubcoreMesh(core_axis_name="c", subcore_axis_name="s")
  out_type = [jax.ShapeDtypeStruct((n_pad, feat), jnp.float32),
              jax.ShapeDtypeStruct((n_pad, feat), jnp.float32)]
  if with_deg:
    out_type.append(jax.ShapeDtypeStruct((NW, n_pad), jnp.float32))
  scratch = [pltpu.VMEM((1, K), jnp.int32)] * (2 * G) \
    + [pltpu.VMEM((1, K), jnp.int32)] * (2 * G) \
    + [
      pltpu.VMEM((G, K, feat), jnp.float32),
      pltpu.VMEM((64, feat), jnp.float32),
      pltpu.VMEM_SHARED((n_pad, feat), jnp.float32),
      pltpu.SemaphoreType.DMA,
      pltpu.SemaphoreType.DMA,
  ] + [pltpu.SemaphoreType.DMA] * G \
    + [pltpu.SemaphoreType.DMA] * G \
    + [pltpu.SemaphoreType.DMA]
  if with_deg:
    scratch.append(pltpu.VMEM((n_pad,), jnp.float32))

  def body(table, src_r, dst_r, acc0, acc1, *rest):
    if with_deg:
      degp = rest[0]
      rest = rest[1:]
    srcb = rest[:2 * G]
    dstb = rest[2 * G:4 * G]
    rows, zbuf, acc_sh = rest[4 * G:4 * G + 3]
    off0 = 4 * G + 3
    isem = rest[off0:off0 + 2]
    gsem = rest[off0 + 2:off0 + 2 + G]
    ssem = rest[off0 + 2 + G:off0 + 2 + 2 * G]
    zsem = rest[off0 + 2 + 2 * G]
    degbuf = rest[off0 + 3 + 2 * G] if with_deg else None
    c = lax.axis_index("c")
    s = lax.axis_index("s")
    wid = c * NS + s
    wbase = wid * epw
    z16 = jnp.zeros((16,), jnp.float32)
    ones = jnp.ones((16,), jnp.float32)
    tailmask = lax.broadcasted_iota(jnp.int32, (16,), 0) >= 8
    gpr = feat // 16

    def zb(t, carry):
      zbuf[t // gpr, pl.ds((t % gpr) * 16, 16)] = z16
      return carry
    lax.fori_loop(0, 64 * gpr, zb, 0)
    for r in range(rps // 64):
      pltpu.make_async_copy(
          zbuf, acc_sh.at[pl.ds(s * rps + r * 64, 64)], zsem).start()
    if with_deg:
      def zd(t, carry):
        degbuf[pl.ds(t * 16, 16)] = z16
        return carry
      lax.fori_loop(0, n_pad // 16, zd, 0)
    for r in range(rps // 64):
      pltpu.make_async_copy(
          zbuf, acc_sh.at[pl.ds(s * rps + r * 64, 64)], zsem).wait()
    plsc.subcore_barrier()

    def fire_idx_group(gi, q):
      off = pl.multiple_of(wbase + gi * (G * K), 8)
      for uu in range(G):
        pltpu.make_async_copy(src_r.at[pl.ds(off + uu * K, K)],
                              srcb[q * G + uu].at[0], isem[q]).start()
        pltpu.make_async_copy(dst_r.at[pl.ds(off + uu * K, K)],
                              dstb[q * G + uu].at[0], isem[q]).start()

    def wait_idx_group(q):
      off = pl.multiple_of(wbase, 8)
      for uu in range(G):
        pltpu.make_async_copy(src_r.at[pl.ds(off + uu * K, K)],
                              srcb[q * G + uu].at[0], isem[q]).wait()
        pltpu.make_async_copy(dst_r.at[pl.ds(off + uu * K, K)],
                              dstb[q * G + uu].at[0], isem[q]).wait()

    def fire_gather(q, u):
      pltpu.make_async_copy(table.at[srcb[q * G + u].at[0]],
                            rows.at[u], gsem[u]).start()

    def wait_gather(u):
      pltpu.make_async_copy(table.at[srcb[u].at[0]],
                            rows.at[u], gsem[u]).wait()

    def fire_scatter(q, u):
      pltpu.async_copy(rows.at[u], acc_sh.at[dstb[q * G + u].at[0]],
                       ssem[u], add=True)

    def wait_scatter(u):
      pltpu.make_async_copy(rows.at[u], acc_sh.at[dstb[u].at[0]],
                            ssem[u]).wait()

    def drain_chunk(q, u):
      wait_gather(u)
      fire_scatter(q, u)
      if with_deg:
        d = dstb[q * G + u]
        plsc.addupdate_scatter(degbuf, [d[0, pl.ds(0, 16)]], ones)
        plsc.addupdate_scatter(degbuf, [d[0, pl.ds(16, 16)]], ones)
        plsc.addupdate_scatter(degbuf, [d[0, pl.ds(24, 16)]], ones,
                               mask=tailmask)

    offp = pl.multiple_of(wbase, 8)
    for uu in range(G):
      pltpu.sync_copy(src_r.at[pl.ds(offp + uu * K, K)], srcb[uu].at[0])
      pltpu.sync_copy(dst_r.at[pl.ds(offp + uu * K, K)], dstb[uu].at[0])
    fire_idx_group(1, 1)
    fire_gather(0, 0)
    fire_gather(0, 1)

    def outer(gg, carry):
      for pp in (0, 1):
        for u in range(G):
          if u < 3:
            w = u + 2
            if pp == 0:
              @pl.when(gg > 0)
              def _(w=w):
                wait_scatter(w)
            else:
              wait_scatter(w)
            fire_gather(pp, w)
          else:
            w = u - 3
            if pp == 0:
              if u == 3:
                wait_idx_group(1)
              wait_scatter(w)
              fire_gather(1, w)
            else:
              @pl.when(gg < nouter - 1)
              def _(w=w, u=u):
                if u == 3:
                  wait_idx_group(0)
                wait_scatter(w)
                fire_gather(0, w)
          drain_chunk(pp, u)
          if u == 4:
            @pl.when(gg < nouter - 1)
            def _(pp=pp):
              fire_idx_group(gg * 2 + 2 + pp, pp)
      return carry

    lax.fori_loop(0, nouter, outer, 0)
    for u in range(G):
      wait_scatter(u)
    plsc.subcore_barrier()

    @pl.when(c == 0)
    def _():
      for r in range(rps // 128):
        sl = pl.ds(s * rps + r * 128, 128)
        pltpu.make_async_copy(acc_sh.at[sl], acc0.at[sl], zsem).start()
      for r in range(rps // 128):
        sl = pl.ds(s * rps + r * 128, 128)
        pltpu.make_async_copy(acc_sh.at[sl], acc0.at[sl], zsem).wait()

    @pl.when(c == 1)
    def _():
      for r in range(rps // 128):
        sl = pl.ds(s * rps + r * 128, 128)
        pltpu.make_async_copy(acc_sh.at[sl], acc1.at[sl], zsem).start()
      for r in range(rps // 128):
        sl = pl.ds(s * rps + r * 128, 128)
        pltpu.make_async_copy(acc_sh.at[sl], acc1.at[sl], zsem).wait()

    if with_deg:
      pltpu.sync_copy(degbuf, degp.at[wid])

  return pl.kernel(body, out_type=tuple(out_type), mesh=mesh,
                   scratch_types=tuple(scratch), compiler_params=_SC_PARAMS)


def _conv2_filtered_kernel(n_pad, feat, n_edges, n_b):
  epw = n_edges // NW
  SP = 2 * n_b
  TRASH = n_b
  R = 5
  assert epw % 16 == 0 and n_b % 128 == 0
  rps = SP // NS
  dps = n_b // NS
  mesh = plsc.VectorSubcoreMesh(core_axis_name="c", subcore_axis_name="s")
  out_type = jax.ShapeDtypeStruct((NC, n_b, feat), jnp.float32)
  scratch = (
      pltpu.VMEM((epw,), jnp.int32),
      pltpu.VMEM((epw,), jnp.int32),
      pltpu.VMEM((n_pad,), jnp.int32),
      pltpu.VMEM((epw + 16,), jnp.int32),
      pltpu.VMEM((epw + 16,), jnp.int32),
      pltpu.VMEM((1, n_b), jnp.int32),
      pltpu.VMEM((R, 16, feat), jnp.float32),
      pltpu.VMEM((64, feat), jnp.float32),
      pltpu.VMEM_SHARED((SP, feat), jnp.float32),
      pltpu.SemaphoreType.DMA,
  ) + (pltpu.SemaphoreType.DMA,) * R \
    + (pltpu.SemaphoreType.DMA,) * R \
    + (pltpu.SemaphoreType.DMA,)

  def body(table, src_r, dst_r, center, bout,
           srcv, dstv, slotmap, fsrc, fslot, cbuf, rows, zbuf, acc_sh,
           esem, *sems):
    gsem = sems[:R]
    ssem = sems[R:2 * R]
    zsem = sems[2 * R]
    c = lax.axis_index("c")
    s = lax.axis_index("s")
    wid = c * NS + s
    wbase = pl.multiple_of(wid * epw, 8)
    z16 = jnp.zeros((16,), jnp.float32)
    zi16 = jnp.zeros((16,), jnp.int32)
    iota16 = lax.broadcasted_iota(jnp.int32, (16,), 0)
    gpr = feat // 16

    pltpu.make_async_copy(src_r.at[pl.ds(wbase, epw)], srcv, esem).start()
    pltpu.make_async_copy(dst_r.at[pl.ds(wbase, epw)], dstv, esem).start()
    pltpu.make_async_copy(center, cbuf.at[0], esem).start()

    def zb(t, carry):
      zbuf[t // gpr, pl.ds((t % gpr) * 16, 16)] = z16
      return carry
    lax.fori_loop(0, 64 * gpr, zb, 0)
    for r in range(rps // 64):
      pltpu.make_async_copy(
          zbuf, acc_sh.at[pl.ds(s * rps + r * 64, 64)], zsem).start()

    trash16 = jnp.full((16,), TRASH, jnp.int32)

    def fill(t, carry):
      slotmap[pl.ds(t * 16, 16)] = trash16
      return carry
    lax.fori_loop(0, n_pad // 16, fill, 0)
    pltpu.make_async_copy(center, cbuf.at[0], esem).wait()

    def scat(g, carry):
      idx = cbuf[0, pl.ds(g * 16, 16)]
      plsc.store_scatter(slotmap, [idx], g * 16 + iota16)
      return carry
    lax.fori_loop(0, n_b // 16, scat, 0)

    pltpu.make_async_copy(src_r.at[pl.ds(wbase, epw)], srcv, esem).wait()
    pltpu.make_async_copy(dst_r.at[pl.ds(wbase, epw)], dstv, esem).wait()
    for r in range(rps // 64):
      pltpu.make_async_copy(
          zbuf, acc_sh.at[pl.ds(s * rps + r * 64, 64)], zsem).wait()
    plsc.subcore_barrier()

    def filt(i, ptr):
      d16 = dstv[pl.ds(i * 16, 16)]
      s16 = srcv[pl.ds(i * 16, 16)]
      sl16 = plsc.load_gather(slotmap, [d16])
      m = sl16 < TRASH
      plsc.store_compressed(fsrc.at[pl.ds(ptr, 16)], s16, mask=m)
      plsc.store_compressed(fslot.at[pl.ds(ptr, 16)], sl16, mask=m)
      cnt = plsc.all_reduce_population_count(m)[0]
      return ptr + cnt
    nf = lax.fori_loop(0, epw // 16, filt, jnp.int32(0))
    fsrc[pl.ds(nf, 16)] = zi16
    fslot[pl.ds(nf, 16)] = trash16
    t2 = (nf + 15) // 16

    def fire_gather(j, u):
      sidx = fsrc[pl.ds(j * 16, 16)]
      pltpu.make_async_copy(table.at[sidx], rows.at[u], gsem[u]).start()

    def wait_gather(u):
      pltpu.make_async_copy(table.at[zi16], rows.at[u], gsem[u]).wait()

    def fire_scatter(j, u):
      didx = fslot[pl.ds(j * 16, 16)]
      pltpu.async_copy(rows.at[u], acc_sh.at[didx], ssem[u], add=True)

    def wait_scatter(u):
      pltpu.make_async_copy(rows.at[u], acc_sh.at[zi16], ssem[u]).wait()

    @pl.when(t2 > 0)
    def _():
      fire_gather(0, 0)

    @pl.when(t2 > 1)
    def _():
      fire_gather(1, 1)

    def pb(t, carry):
      for u in range(R):
        j = t * R + u
        j2 = j + 2
        w = (u + 2) % R

        @pl.when(jnp.logical_and(j2 < t2, j2 >= R))
        def _(w=w):
          wait_scatter(w)

        @pl.when(j2 < t2)
        def _(j2=j2, w=w):
          fire_gather(j2, w)

        @pl.when(j < t2)
        def _(j=j, u=u):
          wait_gather(u)
          fire_scatter(j, u)
      return carry
    lax.fori_loop(0, (t2 + R - 1) // R, pb, 0)
    for u in range(R):
      @pl.when(u < t2)
      def _(u=u):
        wait_scatter(u)
    plsc.subcore_barrier()

    pltpu.sync_copy(acc_sh.at[pl.ds(s * dps, dps)],
                    bout.at[c].at[pl.ds(s * dps, dps)])

  return pl.kernel(body, out_type=out_type, mesh=mesh, scratch_types=scratch,
                   compiler_params=_SC_PARAMS)


def _center_gather_kernel(n_pad, feat, n_b):
  bpw = n_b // NW
  TRASH = n_b
  assert bpw * NW == n_b and bpw % 16 == 0
  mesh = plsc.VectorSubcoreMesh(core_axis_name="c", subcore_axis_name="s")
  out_type = (jax.ShapeDtypeStruct((n_b, feat), jnp.float32),
              jax.ShapeDtypeStruct((n_b, feat), jnp.float32),
              jax.ShapeDtypeStruct((n_b,), jnp.float32))
  scratch = (
      pltpu.VMEM((1, n_b), jnp.int32),
      pltpu.VMEM((n_pad,), jnp.int32),
      pltpu.VMEM((bpw, feat), jnp.float32),
      pltpu.VMEM((bpw, feat), jnp.float32),
      pltpu.VMEM((n_pad,), jnp.float32),
      pltpu.VMEM((bpw,), jnp.float32),
  ) + tuple(pltpu.VMEM((1, 16), jnp.int32) for _ in range(bpw // 16)) + (
      pltpu.SemaphoreType.DMA,
      pltpu.SemaphoreType.DMA,
  )

  def body(b0, b1, invdeg, center, r0, r1, idegc, *rest):
    ngr = bpw // 16
    cbuf, slotmap, rows0, rows1, degv, degc = rest[:6]
    slb = rest[6:6 + ngr]
    esem, gsem = rest[6 + ngr:8 + ngr]
    c = lax.axis_index("c")
    s = lax.axis_index("s")
    base = (c * NS + s) * bpw
    iota16 = lax.broadcasted_iota(jnp.int32, (16,), 0)
    trash16 = jnp.full((16,), TRASH, jnp.int32)

    pltpu.make_async_copy(center, cbuf.at[0], esem).start()
    pltpu.make_async_copy(invdeg, degv, esem).start()

    def fill(t, carry):
      slotmap[pl.ds(t * 16, 16)] = trash16
      return carry
    lax.fori_loop(0, n_pad // 16, fill, 0)
    pltpu.make_async_copy(center, cbuf.at[0], esem).wait()

    def scat(g, carry):
      idx = cbuf[0, pl.ds(g * 16, 16)]
      plsc.store_scatter(slotmap, [idx], g * 16 + iota16)
      return carry
    lax.fori_loop(0, n_b // 16, scat, 0)

    pltpu.make_async_copy(invdeg, degv, esem).wait()
    for g in range(ngr):
      cidx = cbuf[0, pl.ds(base + g * 16, 16)]
      sl16 = plsc.load_gather(slotmap, [cidx])
      slb[g][0, pl.ds(0, 16)] = sl16
      pltpu.make_async_copy(b0.at[slb[g].at[0]],
                            rows0.at[pl.ds(g * 16, 16)], gsem).start()
      pltpu.make_async_copy(b1.at[slb[g].at[0]],
                            rows1.at[pl.ds(g * 16, 16)], gsem).start()
      degc[pl.ds(g * 16, 16)] = plsc.load_gather(degv, [cidx])
    for g in range(ngr):
      pltpu.make_async_copy(b0.at[slb[g].at[0]],
                            rows0.at[pl.ds(g * 16, 16)], gsem).wait()
      pltpu.make_async_copy(b1.at[slb[g].at[0]],
                            rows1.at[pl.ds(g * 16, 16)], gsem).wait()
    pltpu.sync_copy(rows0, r0.at[pl.ds(base, bpw)])
    pltpu.sync_copy(rows1, r1.at[pl.ds(base, bpw)])
    pltpu.sync_copy(degc, idegc.at[pl.ds(base, bpw)])

  return pl.kernel(body, out_type=out_type, mesh=mesh, scratch_types=scratch,
                   compiler_params=_SC_PARAMS)


def _conv_dense(n_pad, d_in, d_out):
  blk = 1280
  grid = (n_pad // blk,)

  def body(a0, a1, degp, w, bb, h, invd):
    deg = jnp.sum(degp[...], axis=1)
    inv = 1.0 / jnp.maximum(deg, 1.0)
    agg = (a0[...] + a1[...]) * inv[:, None]
    h[...] = jnp.maximum(
        jnp.dot(agg, w[...], preferred_element_type=jnp.float32) + bb[...], 0.0)
    invd[...] = inv[:, None]

  return pl.pallas_call(
      body,
      grid=grid,
      in_specs=[
          pl.BlockSpec((blk, d_in), lambda i: (i, 0)),
          pl.BlockSpec((blk, d_in), lambda i: (i, 0)),
          pl.BlockSpec((blk, NW), lambda i: (i, 0)),
          pl.BlockSpec((d_in, d_out), lambda i: (0, 0)),
          pl.BlockSpec((1, d_out), lambda i: (0, 0)),
      ],
      out_specs=[
          pl.BlockSpec((blk, d_out), lambda i: (i, 0)),
          pl.BlockSpec((blk, 1), lambda i: (i, 0)),
      ],
      out_shape=[
          jax.ShapeDtypeStruct((n_pad, d_out), jnp.float32),
          jax.ShapeDtypeStruct((n_pad, 1), jnp.float32),
      ],
  )


def _final_kernel(h, n_b, h2):
  def body(r0c, r1c, idc, r0t, r1t, idt, cw2, cb2, tw2, tb2,
           pw1, pb1, pw2, pb2, zp, zt):
    zc = jnp.dot((r0c[...] + r1c[...]) * idc[...], cw2[...],
                 preferred_element_type=jnp.float32) + cb2[...]
    hid = jnp.maximum(
        jnp.dot(zc, pw1[...], preferred_element_type=jnp.float32) + pb1[...],
        0.0)
    zp[...] = jnp.dot(hid, pw2[...],
                      preferred_element_type=jnp.float32) + pb2[...]
    zt[...] = jnp.dot((r0t[...] + r1t[...]) * idt[...], tw2[...],
                      preferred_element_type=jnp.float32) + tb2[...]

  return pl.pallas_call(
      body,
      out_shape=[jax.ShapeDtypeStruct((n_b, h), jnp.float32),
                 jax.ShapeDtypeStruct((n_b, h), jnp.float32)])


def kernel(x_context, edge_index_context, center_mask_context,
           x_target, edge_index_target, center_mask_target,
           cW1, cb1, cW2, cb2, tW1, tb1, tW2, tb2,
           pW1, pb1, pW2, pb2):
  n, d = x_context.shape
  e = edge_index_context.shape[1]
  nb = center_mask_context.shape[0]
  h = cW1.shape[1]
  h2 = pW1.shape[1]
  n_pad = -(-n // (NS * 128)) * (NS * 128)

  seg_deg = _seg_sum_kernel(n_pad, d, e, True)
  seg2 = _conv2_filtered_kernel(n_pad, h, e, nb)
  conv = _conv_dense(n_pad, d, h)
  gath = _center_gather_kernel(n_pad, h, nb)
  fin = _final_kernel(h, nb, h2)

  def enc(x, ei, ctr, w1, bias1):
    src, dst = ei[0], ei[1]
    a0, a1, degp = seg_deg(x, src, dst)
    hdn, invd = conv(a0, a1, degp.T, w1, bias1.reshape(1, h))
    bout = seg2(hdn, src, dst, ctr)
    r0, r1, idc = gath(bout[0], bout[1], invd.reshape(-1), ctr)
    return r0, r1, idc.reshape(nb, 1)

  r0c, r1c, idc = enc(x_context, edge_index_context, center_mask_context,
                      cW1, cb1)
  r0t, r1t, idt = enc(x_target, edge_index_target, center_mask_target,
                      tW1, tb1)
  zp, zt = fin(r0c, r1c, idc, r0t, r1t, idt,
               cW2, cb2.reshape(1, h), tW2, tb2.reshape(1, h),
               pW1, pb1.reshape(1, h2), pW2, pb2.reshape(1, h))
  return (zp, zt)


# speedup vs baseline: 14.5280x; 1.0107x over previous
import jax
import jax.numpy as jnp
from jax import lax
from jax.experimental import pallas as pl
from jax.experimental.pallas import tpu as pltpu
from jax.experimental.pallas import tpu_sc as plsc

NC = 2
NS = 16
NW = NC * NS

_SC_PARAMS = pltpu.CompilerParams(needs_layout_passes=False)


_K = 40
_G = 5


def _seg_sum_kernel(n_pad, feat, n_edges, with_deg):
  K, G = _K, _G
  epw = n_edges // NW
  nchunk = epw // K
  ng = nchunk // G
  nouter = ng // 2
  assert epw * NW == n_edges and K * nchunk == epw and G * ng == nchunk
  assert ng % 2 == 0 and n_pad % (NS * 128) == 0
  rps = n_pad // NS
  mesh = plsc.VectorSubcoreMesh(core_axis_name="c", subcore_axis_name="s")
  out_type = [jax.ShapeDtypeStruct((n_pad, feat), jnp.float32),
              jax.ShapeDtypeStruct((n_pad, feat), jnp.float32)]
  if with_deg:
    out_type.append(jax.ShapeDtypeStruct((NW, n_pad), jnp.float32))
  scratch = [pltpu.VMEM((1, K), jnp.int32)] * (2 * G) \
    + [pltpu.VMEM((1, K), jnp.int32)] * (2 * G) \
    + [
      pltpu.VMEM((G, K, feat), jnp.float32),
      pltpu.VMEM((64, feat), jnp.float32),
      pltpu.VMEM_SHARED((n_pad, feat), jnp.float32),
      pltpu.SemaphoreType.DMA,
      pltpu.SemaphoreType.DMA,
  ] + [pltpu.SemaphoreType.DMA] * G \
    + [pltpu.SemaphoreType.DMA] * G \
    + [pltpu.SemaphoreType.DMA]
  if with_deg:
    scratch.append(pltpu.VMEM((n_pad,), jnp.float32))

  def body(table, src_r, dst_r, acc0, acc1, *rest):
    if with_deg:
      degp = rest[0]
      rest = rest[1:]
    srcb = rest[:2 * G]
    dstb = rest[2 * G:4 * G]
    rows, zbuf, acc_sh = rest[4 * G:4 * G + 3]
    off0 = 4 * G + 3
    isem = rest[off0:off0 + 2]
    gsem = rest[off0 + 2:off0 + 2 + G]
    ssem = rest[off0 + 2 + G:off0 + 2 + 2 * G]
    zsem = rest[off0 + 2 + 2 * G]
    degbuf = rest[off0 + 3 + 2 * G] if with_deg else None
    c = lax.axis_index("c")
    s = lax.axis_index("s")
    wid = c * NS + s
    wbase = wid * epw
    z16 = jnp.zeros((16,), jnp.float32)
    ones = jnp.ones((16,), jnp.float32)
    tailmask = lax.broadcasted_iota(jnp.int32, (16,), 0) >= 8
    gpr = feat // 16

    def zb(t, carry):
      zbuf[t // gpr, pl.ds((t % gpr) * 16, 16)] = z16
      return carry
    lax.fori_loop(0, 64 * gpr, zb, 0)
    for r in range(rps // 64):
      pltpu.make_async_copy(
          zbuf, acc_sh.at[pl.ds(s * rps + r * 64, 64)], zsem).start()
    if with_deg:
      def zd(t, carry):
        degbuf[pl.ds(t * 16, 16)] = z16
        return carry
      lax.fori_loop(0, n_pad // 16, zd, 0)
    for r in range(rps // 64):
      pltpu.make_async_copy(
          zbuf, acc_sh.at[pl.ds(s * rps + r * 64, 64)], zsem).wait()
    plsc.subcore_barrier()

    def fire_idx_group(gi, q):
      off = pl.multiple_of(wbase + gi * (G * K), 8)
      for uu in range(G):
        pltpu.make_async_copy(src_r.at[pl.ds(off + uu * K, K)],
                              srcb[q * G + uu].at[0], isem[q]).start()
        pltpu.make_async_copy(dst_r.at[pl.ds(off + uu * K, K)],
                              dstb[q * G + uu].at[0], isem[q]).start()

    def wait_idx_group(q):
      off = pl.multiple_of(wbase, 8)
      for uu in range(G):
        pltpu.make_async_copy(src_r.at[pl.ds(off + uu * K, K)],
                              srcb[q * G + uu].at[0], isem[q]).wait()
        pltpu.make_async_copy(dst_r.at[pl.ds(off + uu * K, K)],
                              dstb[q * G + uu].at[0], isem[q]).wait()

    def fire_gather(q, u):
      pltpu.make_async_copy(table.at[srcb[q * G + u].at[0]],
                            rows.at[u], gsem[u]).start()

    def wait_gather(u):
      pltpu.make_async_copy(table.at[srcb[u].at[0]],
                            rows.at[u], gsem[u]).wait()

    def fire_scatter(q, u):
      pltpu.async_copy(rows.at[u], acc_sh.at[dstb[q * G + u].at[0]],
                       ssem[u], add=True)

    def wait_scatter(u):
      pltpu.make_async_copy(rows.at[u], acc_sh.at[dstb[u].at[0]],
                            ssem[u]).wait()

    def drain_chunk(q, u):
      wait_gather(u)
      fire_scatter(q, u)
      if with_deg:
        d = dstb[q * G + u]
        plsc.addupdate_scatter(degbuf, [d[0, pl.ds(0, 16)]], ones)
        plsc.addupdate_scatter(degbuf, [d[0, pl.ds(16, 16)]], ones)
        plsc.addupdate_scatter(degbuf, [d[0, pl.ds(24, 16)]], ones,
                               mask=tailmask)

    offp = pl.multiple_of(wbase, 8)
    for uu in range(G):
      pltpu.sync_copy(src_r.at[pl.ds(offp + uu * K, K)], srcb[uu].at[0])
      pltpu.sync_copy(dst_r.at[pl.ds(offp + uu * K, K)], dstb[uu].at[0])
    fire_idx_group(1, 1)
    fire_gather(0, 0)
    fire_gather(0, 1)

    def outer(gg, carry):
      for pp in (0, 1):
        for u in range(G):
          if u < 3:
            w = u + 2
            if pp == 0:
              @pl.when(gg > 0)
              def _(w=w):
                wait_scatter(w)
            else:
              wait_scatter(w)
            fire_gather(pp, w)
          else:
            w = u - 3
            if pp == 0:
              if u == 3:
                wait_idx_group(1)
              wait_scatter(w)
              fire_gather(1, w)
            else:
              @pl.when(gg < nouter - 1)
              def _(w=w, u=u):
                if u == 3:
                  wait_idx_group(0)
                wait_scatter(w)
                fire_gather(0, w)
          drain_chunk(pp, u)
          if u == 4:
            @pl.when(gg < nouter - 1)
            def _(pp=pp):
              fire_idx_group(gg * 2 + 2 + pp, pp)
      return carry

    lax.fori_loop(0, nouter, outer, 0)
    for u in range(G):
      wait_scatter(u)
    plsc.subcore_barrier()

    @pl.when(c == 0)
    def _():
      for r in range(rps // 128):
        sl = pl.ds(s * rps + r * 128, 128)
        pltpu.make_async_copy(acc_sh.at[sl], acc0.at[sl], zsem).start()
      for r in range(rps // 128):
        sl = pl.ds(s * rps + r * 128, 128)
        pltpu.make_async_copy(acc_sh.at[sl], acc0.at[sl], zsem).wait()

    @pl.when(c == 1)
    def _():
      for r in range(rps // 128):
        sl = pl.ds(s * rps + r * 128, 128)
        pltpu.make_async_copy(acc_sh.at[sl], acc1.at[sl], zsem).start()
      for r in range(rps // 128):
        sl = pl.ds(s * rps + r * 128, 128)
        pltpu.make_async_copy(acc_sh.at[sl], acc1.at[sl], zsem).wait()

    if with_deg:
      pltpu.sync_copy(degbuf, degp.at[wid])

  return pl.kernel(body, out_type=tuple(out_type), mesh=mesh,
                   scratch_types=tuple(scratch), compiler_params=_SC_PARAMS)


def _conv2_filtered_kernel(n_pad, feat, n_edges, n_b):
  epw = n_edges // NW
  SP = 2 * n_b
  TRASH = n_b
  R = 5
  assert epw % 16 == 0 and n_b % 128 == 0
  rps = SP // NS
  dps = n_b // NS
  mesh = plsc.VectorSubcoreMesh(core_axis_name="c", subcore_axis_name="s")
  out_type = jax.ShapeDtypeStruct((NC, n_b, feat), jnp.float32)
  scratch = (
      pltpu.VMEM((epw,), jnp.int32),
      pltpu.VMEM((epw,), jnp.int32),
      pltpu.VMEM((n_pad,), jnp.int32),
      pltpu.VMEM((epw + 16,), jnp.int32),
      pltpu.VMEM((epw + 16,), jnp.int32),
      pltpu.VMEM((1, n_b), jnp.int32),
      pltpu.VMEM((R, 16, feat), jnp.float32),
      pltpu.VMEM((64, feat), jnp.float32),
      pltpu.VMEM_SHARED((SP, feat), jnp.float32),
      pltpu.SemaphoreType.DMA,
  ) + (pltpu.SemaphoreType.DMA,) * R \
    + (pltpu.SemaphoreType.DMA,) * R \
    + (pltpu.SemaphoreType.DMA,)

  def body(table, src_r, dst_r, center, bout,
           srcv, dstv, slotmap, fsrc, fslot, cbuf, rows, zbuf, acc_sh,
           esem, *sems):
    gsem = sems[:R]
    ssem = sems[R:2 * R]
    zsem = sems[2 * R]
    c = lax.axis_index("c")
    s = lax.axis_index("s")
    wid = c * NS + s
    wbase = pl.multiple_of(wid * epw, 8)
    z16 = jnp.zeros((16,), jnp.float32)
    zi16 = jnp.zeros((16,), jnp.int32)
    iota16 = lax.broadcasted_iota(jnp.int32, (16,), 0)
    gpr = feat // 16

    pltpu.make_async_copy(src_r.at[pl.ds(wbase, epw)], srcv, esem).start()
    pltpu.make_async_copy(dst_r.at[pl.ds(wbase, epw)], dstv, esem).start()
    pltpu.make_async_copy(center, cbuf.at[0], esem).start()

    def zb(t, carry):
      zbuf[t // gpr, pl.ds((t % gpr) * 16, 16)] = z16
      return carry
    lax.fori_loop(0, 64 * gpr, zb, 0)
    for r in range(rps // 64):
      pltpu.make_async_copy(
          zbuf, acc_sh.at[pl.ds(s * rps + r * 64, 64)], zsem).start()

    trash16 = jnp.full((16,), TRASH, jnp.int32)

    def fill(t, carry):
      slotmap[pl.ds(t * 16, 16)] = trash16
      return carry
    lax.fori_loop(0, n_pad // 16, fill, 0)
    pltpu.make_async_copy(center, cbuf.at[0], esem).wait()

    def scat(g, carry):
      idx = cbuf[0, pl.ds(g * 16, 16)]
      plsc.store_scatter(slotmap, [idx], g * 16 + iota16)
      return carry
    lax.fori_loop(0, n_b // 16, scat, 0)

    pltpu.make_async_copy(src_r.at[pl.ds(wbase, epw)], srcv, esem).wait()
    pltpu.make_async_copy(dst_r.at[pl.ds(wbase, epw)], dstv, esem).wait()
    for r in range(rps // 64):
      pltpu.make_async_copy(
          zbuf, acc_sh.at[pl.ds(s * rps + r * 64, 64)], zsem).wait()
    plsc.subcore_barrier()

    def filt(i, ptr):
      d16 = dstv[pl.ds(i * 16, 16)]
      s16 = srcv[pl.ds(i * 16, 16)]
      sl16 = plsc.load_gather(slotmap, [d16])
      m = sl16 < TRASH
      plsc.store_compressed(fsrc.at[pl.ds(ptr, 16)], s16, mask=m)
      plsc.store_compressed(fslot.at[pl.ds(ptr, 16)], sl16, mask=m)
      cnt = plsc.all_reduce_population_count(m)[0]
      return ptr + cnt
    nf = lax.fori_loop(0, epw // 16, filt, jnp.int32(0))
    fsrc[pl.ds(nf, 16)] = zi16
    fslot[pl.ds(nf, 16)] = trash16
    t2 = (nf + 15) // 16

    def fire_gather(j, u):
      sidx = fsrc[pl.ds(j * 16, 16)]
      pltpu.make_async_copy(table.at[sidx], rows.at[u], gsem[u]).start()

    def wait_gather(u):
      pltpu.make_async_copy(table.at[zi16], rows.at[u], gsem[u]).wait()

    def fire_scatter(j, u):
      didx = fslot[pl.ds(j * 16, 16)]
      pltpu.async_copy(rows.at[u], acc_sh.at[didx], ssem[u], add=True)

    def wait_scatter(u):
      pltpu.make_async_copy(rows.at[u], acc_sh.at[zi16], ssem[u]).wait()

    for jp in range(3):
      @pl.when(jp < t2)
      def _(jp=jp):
        fire_gather(jp, jp)

    def pb(t, carry):
      for u in range(R):
        j = t * R + u
        j2 = j + 3
        w = (u + 3) % R

        @pl.when(jnp.logical_and(j2 < t2, j2 >= R))
        def _(w=w):
          wait_scatter(w)

        @pl.when(j2 < t2)
        def _(j2=j2, w=w):
          fire_gather(j2, w)

        @pl.when(j < t2)
        def _(j=j, u=u):
          wait_gather(u)
          fire_scatter(j, u)
      return carry
    lax.fori_loop(0, (t2 + R - 1) // R, pb, 0)
    for u in range(R):
      @pl.when(u < t2)
      def _(u=u):
        wait_scatter(u)
    plsc.subcore_barrier()

    pltpu.sync_copy(acc_sh.at[pl.ds(s * dps, dps)],
                    bout.at[c].at[pl.ds(s * dps, dps)])

  return pl.kernel(body, out_type=out_type, mesh=mesh, scratch_types=scratch,
                   compiler_params=_SC_PARAMS)


def _center_gather_kernel(n_pad, feat, n_b):
  bpw = n_b // NW
  TRASH = n_b
  assert bpw * NW == n_b and bpw % 16 == 0
  mesh = plsc.VectorSubcoreMesh(core_axis_name="c", subcore_axis_name="s")
  out_type = (jax.ShapeDtypeStruct((n_b, feat), jnp.float32),
              jax.ShapeDtypeStruct((n_b, feat), jnp.float32),
              jax.ShapeDtypeStruct((n_b,), jnp.float32))
  scratch = (
      pltpu.VMEM((1, n_b), jnp.int32),
      pltpu.VMEM((n_pad,), jnp.int32),
      pltpu.VMEM((bpw, feat), jnp.float32),
      pltpu.VMEM((bpw, feat), jnp.float32),
      pltpu.VMEM((n_pad,), jnp.float32),
      pltpu.VMEM((bpw,), jnp.float32),
  ) + tuple(pltpu.VMEM((1, 16), jnp.int32) for _ in range(bpw // 16)) + (
      pltpu.SemaphoreType.DMA,
      pltpu.SemaphoreType.DMA,
  )

  def body(b0, b1, invdeg, center, r0, r1, idegc, *rest):
    ngr = bpw // 16
    cbuf, slotmap, rows0, rows1, degv, degc = rest[:6]
    slb = rest[6:6 + ngr]
    esem, gsem = rest[6 + ngr:8 + ngr]
    c = lax.axis_index("c")
    s = lax.axis_index("s")
    base = (c * NS + s) * bpw
    iota16 = lax.broadcasted_iota(jnp.int32, (16,), 0)
    trash16 = jnp.full((16,), TRASH, jnp.int32)

    pltpu.make_async_copy(center, cbuf.at[0], esem).start()
    pltpu.make_async_copy(invdeg, degv, esem).start()

    def fill(t, carry):
      slotmap[pl.ds(t * 16, 16)] = trash16
      return carry
    lax.fori_loop(0, n_pad // 16, fill, 0)
    pltpu.make_async_copy(center, cbuf.at[0], esem).wait()

    def scat(g, carry):
      idx = cbuf[0, pl.ds(g * 16, 16)]
      plsc.store_scatter(slotmap, [idx], g * 16 + iota16)
      return carry
    lax.fori_loop(0, n_b // 16, scat, 0)

    pltpu.make_async_copy(invdeg, degv, esem).wait()
    for g in range(ngr):
      cidx = cbuf[0, pl.ds(base + g * 16, 16)]
      sl16 = plsc.load_gather(slotmap, [cidx])
      slb[g][0, pl.ds(0, 16)] = sl16
      pltpu.make_async_copy(b0.at[slb[g].at[0]],
                            rows0.at[pl.ds(g * 16, 16)], gsem).start()
      pltpu.make_async_copy(b1.at[slb[g].at[0]],
                            rows1.at[pl.ds(g * 16, 16)], gsem).start()
      degc[pl.ds(g * 16, 16)] = plsc.load_gather(degv, [cidx])
    for g in range(ngr):
      pltpu.make_async_copy(b0.at[slb[g].at[0]],
                            rows0.at[pl.ds(g * 16, 16)], gsem).wait()
      pltpu.make_async_copy(b1.at[slb[g].at[0]],
                            rows1.at[pl.ds(g * 16, 16)], gsem).wait()
    pltpu.sync_copy(rows0, r0.at[pl.ds(base, bpw)])
    pltpu.sync_copy(rows1, r1.at[pl.ds(base, bpw)])
    pltpu.sync_copy(degc, idegc.at[pl.ds(base, bpw)])

  return pl.kernel(body, out_type=out_type, mesh=mesh, scratch_types=scratch,
                   compiler_params=_SC_PARAMS)


def _conv_dense(n_pad, d_in, d_out):
  blk = 1280
  grid = (n_pad // blk,)

  def body(a0, a1, degp, w, bb, h, invd):
    deg = jnp.sum(degp[...], axis=1)
    inv = 1.0 / jnp.maximum(deg, 1.0)
    agg = (a0[...] + a1[...]) * inv[:, None]
    h[...] = jnp.maximum(
        jnp.dot(agg, w[...], preferred_element_type=jnp.float32) + bb[...], 0.0)
    invd[...] = inv[:, None]

  return pl.pallas_call(
      body,
      grid=grid,
      in_specs=[
          pl.BlockSpec((blk, d_in), lambda i: (i, 0)),
          pl.BlockSpec((blk, d_in), lambda i: (i, 0)),
          pl.BlockSpec((blk, NW), lambda i: (i, 0)),
          pl.BlockSpec((d_in, d_out), lambda i: (0, 0)),
          pl.BlockSpec((1, d_out), lambda i: (0, 0)),
      ],
      out_specs=[
          pl.BlockSpec((blk, d_out), lambda i: (i, 0)),
          pl.BlockSpec((blk, 1), lambda i: (i, 0)),
      ],
      out_shape=[
          jax.ShapeDtypeStruct((n_pad, d_out), jnp.float32),
          jax.ShapeDtypeStruct((n_pad, 1), jnp.float32),
      ],
  )


def _final_kernel(h, n_b, h2):
  def body(r0c, r1c, idc, r0t, r1t, idt, cw2, cb2, tw2, tb2,
           pw1, pb1, pw2, pb2, zp, zt):
    zc = jnp.dot((r0c[...] + r1c[...]) * idc[...], cw2[...],
                 preferred_element_type=jnp.float32) + cb2[...]
    hid = jnp.maximum(
        jnp.dot(zc, pw1[...], preferred_element_type=jnp.float32) + pb1[...],
        0.0)
    zp[...] = jnp.dot(hid, pw2[...],
                      preferred_element_type=jnp.float32) + pb2[...]
    zt[...] = jnp.dot((r0t[...] + r1t[...]) * idt[...], tw2[...],
                      preferred_element_type=jnp.float32) + tb2[...]

  return pl.pallas_call(
      body,
      out_shape=[jax.ShapeDtypeStruct((n_b, h), jnp.float32),
                 jax.ShapeDtypeStruct((n_b, h), jnp.float32)])


def kernel(x_context, edge_index_context, center_mask_context,
           x_target, edge_index_target, center_mask_target,
           cW1, cb1, cW2, cb2, tW1, tb1, tW2, tb2,
           pW1, pb1, pW2, pb2):
  n, d = x_context.shape
  e = edge_index_context.shape[1]
  nb = center_mask_context.shape[0]
  h = cW1.shape[1]
  h2 = pW1.shape[1]
  n_pad = -(-n // (NS * 128)) * (NS * 128)

  seg_deg = _seg_sum_kernel(n_pad, d, e, True)
  seg2 = _conv2_filtered_kernel(n_pad, h, e, nb)
  conv = _conv_dense(n_pad, d, h)
  gath = _center_gather_kernel(n_pad, h, nb)
  fin = _final_kernel(h, nb, h2)

  def enc(x, ei, ctr, w1, bias1):
    src, dst = ei[0], ei[1]
    a0, a1, degp = seg_deg(x, src, dst)
    hdn, invd = conv(a0, a1, degp.T, w1, bias1.reshape(1, h))
    bout = seg2(hdn, src, dst, ctr)
    r0, r1, idc = gath(bout[0], bout[1], invd.reshape(-1), ctr)
    return r0, r1, idc.reshape(nb, 1)

  r0c, r1c, idc = enc(x_context, edge_index_context, center_mask_context,
                      cW1, cb1)
  r0t, r1t, idt = enc(x_target, edge_index_target, center_mask_target,
                      tW1, tb1)
  zp, zt = fin(r0c, r1c, idc, r0t, r1t, idt,
               cW2, cb2.reshape(1, h), tW2, tb2.reshape(1, h),
               pW1, pb1.reshape(1, h2), pW2, pb2.reshape(1, h))
  return (zp, zt)


# speedup vs baseline: 14.6417x; 1.0078x over previous
import jax
import jax.numpy as jnp
from jax import lax
from jax.experimental import pallas as pl
from jax.experimental.pallas import tpu as pltpu
from jax.experimental.pallas import tpu_sc as plsc

NC = 2
NS = 16
NW = NC * NS

_SC_PARAMS = pltpu.CompilerParams(needs_layout_passes=False)


_K = 40
_G = 5


def _seg_sum_kernel(n_pad, feat, n_edges, with_deg):
  K, G = _K, _G
  epw = n_edges // NW
  nchunk = epw // K
  ng = nchunk // G
  nouter = ng // 2
  assert epw * NW == n_edges and K * nchunk == epw and G * ng == nchunk
  assert ng % 2 == 0 and n_pad % (NS * 128) == 0
  rps = n_pad // NS
  mesh = plsc.VectorSubcoreMesh(core_axis_name="c", subcore_axis_name="s")
  out_type = [jax.ShapeDtypeStruct((n_pad, feat), jnp.float32),
              jax.ShapeDtypeStruct((n_pad, feat), jnp.float32)]
  if with_deg:
    out_type.append(jax.ShapeDtypeStruct((NW, n_pad), jnp.float32))
  scratch = [pltpu.VMEM((1, K), jnp.int32)] * (2 * G) \
    + [pltpu.VMEM((1, K), jnp.int32)] * (2 * G) \
    + [
      pltpu.VMEM((G, K, feat), jnp.float32),
      pltpu.VMEM((64, feat), jnp.float32),
      pltpu.VMEM_SHARED((n_pad, feat), jnp.float32),
      pltpu.SemaphoreType.DMA,
      pltpu.SemaphoreType.DMA,
  ] + [pltpu.SemaphoreType.DMA] * G \
    + [pltpu.SemaphoreType.DMA] * G \
    + [pltpu.SemaphoreType.DMA]
  if with_deg:
    scratch.append(pltpu.VMEM((n_pad,), jnp.float32))

  def body(table, src_r, dst_r, acc0, acc1, *rest):
    if with_deg:
      degp = rest[0]
      rest = rest[1:]
    srcb = rest[:2 * G]
    dstb = rest[2 * G:4 * G]
    rows, zbuf, acc_sh = rest[4 * G:4 * G + 3]
    off0 = 4 * G + 3
    isem = rest[off0:off0 + 2]
    gsem = rest[off0 + 2:off0 + 2 + G]
    ssem = rest[off0 + 2 + G:off0 + 2 + 2 * G]
    zsem = rest[off0 + 2 + 2 * G]
    degbuf = rest[off0 + 3 + 2 * G] if with_deg else None
    c = lax.axis_index("c")
    s = lax.axis_index("s")
    wid = c * NS + s
    wbase = wid * epw
    z16 = jnp.zeros((16,), jnp.float32)
    ones = jnp.ones((16,), jnp.float32)
    tailmask = lax.broadcasted_iota(jnp.int32, (16,), 0) >= 8
    gpr = feat // 16

    def zb(t, carry):
      zbuf[t // gpr, pl.ds((t % gpr) * 16, 16)] = z16
      return carry
    lax.fori_loop(0, 64 * gpr, zb, 0)
    for r in range(rps // 64):
      pltpu.make_async_copy(
          zbuf, acc_sh.at[pl.ds(s * rps + r * 64, 64)], zsem).start()
    if with_deg:
      def zd(t, carry):
        degbuf[pl.ds(t * 16, 16)] = z16
        return carry
      lax.fori_loop(0, n_pad // 16, zd, 0)
    for r in range(rps // 64):
      pltpu.make_async_copy(
          zbuf, acc_sh.at[pl.ds(s * rps + r * 64, 64)], zsem).wait()
    plsc.subcore_barrier()

    def fire_idx_group(gi, q):
      off = pl.multiple_of(wbase + gi * (G * K), 8)
      for uu in range(G):
        pltpu.make_async_copy(src_r.at[pl.ds(off + uu * K, K)],
                              srcb[q * G + uu].at[0], isem[q]).start()
        pltpu.make_async_copy(dst_r.at[pl.ds(off + uu * K, K)],
                              dstb[q * G + uu].at[0], isem[q]).start()

    def wait_idx_group(q):
      off = pl.multiple_of(wbase, 8)
      for uu in range(G):
        pltpu.make_async_copy(src_r.at[pl.ds(off + uu * K, K)],
                              srcb[q * G + uu].at[0], isem[q]).wait()
        pltpu.make_async_copy(dst_r.at[pl.ds(off + uu * K, K)],
                              dstb[q * G + uu].at[0], isem[q]).wait()

    def fire_gather(q, u):
      pltpu.make_async_copy(table.at[srcb[q * G + u].at[0]],
                            rows.at[u], gsem[u]).start()

    def wait_gather(u):
      pltpu.make_async_copy(table.at[srcb[u].at[0]],
                            rows.at[u], gsem[u]).wait()

    def fire_scatter(q, u):
      pltpu.async_copy(rows.at[u], acc_sh.at[dstb[q * G + u].at[0]],
                       ssem[u], add=True)

    def wait_scatter(u):
      pltpu.make_async_copy(rows.at[u], acc_sh.at[dstb[u].at[0]],
                            ssem[u]).wait()

    def drain_chunk(q, u):
      wait_gather(u)
      fire_scatter(q, u)
      if with_deg:
        d = dstb[q * G + u]
        plsc.addupdate_scatter(degbuf, [d[0, pl.ds(0, 16)]], ones)
        plsc.addupdate_scatter(degbuf, [d[0, pl.ds(16, 16)]], ones)
        plsc.addupdate_scatter(degbuf, [d[0, pl.ds(24, 16)]], ones,
                               mask=tailmask)

    offp = pl.multiple_of(wbase, 8)
    for uu in range(G):
      pltpu.sync_copy(src_r.at[pl.ds(offp + uu * K, K)], srcb[uu].at[0])
      pltpu.sync_copy(dst_r.at[pl.ds(offp + uu * K, K)], dstb[uu].at[0])
    fire_idx_group(1, 1)
    fire_gather(0, 0)
    fire_gather(0, 1)

    def outer(gg, carry):
      for pp in (0, 1):
        for u in range(G):
          if u < 3:
            w = u + 2
            if pp == 0:
              @pl.when(gg > 0)
              def _(w=w):
                wait_scatter(w)
            else:
              wait_scatter(w)
            fire_gather(pp, w)
          else:
            w = u - 3
            if pp == 0:
              if u == 3:
                wait_idx_group(1)
              wait_scatter(w)
              fire_gather(1, w)
            else:
              @pl.when(gg < nouter - 1)
              def _(w=w, u=u):
                if u == 3:
                  wait_idx_group(0)
                wait_scatter(w)
                fire_gather(0, w)
          drain_chunk(pp, u)
          if u == 4:
            @pl.when(gg < nouter - 1)
            def _(pp=pp):
              fire_idx_group(gg * 2 + 2 + pp, pp)
      return carry

    lax.fori_loop(0, nouter, outer, 0)
    for u in range(G):
      wait_scatter(u)
    plsc.subcore_barrier()

    @pl.when(c == 0)
    def _():
      for r in range(rps // 128):
        sl = pl.ds(s * rps + r * 128, 128)
        pltpu.make_async_copy(acc_sh.at[sl], acc0.at[sl], zsem).start()
      for r in range(rps // 128):
        sl = pl.ds(s * rps + r * 128, 128)
        pltpu.make_async_copy(acc_sh.at[sl], acc0.at[sl], zsem).wait()

    @pl.when(c == 1)
    def _():
      for r in range(rps // 128):
        sl = pl.ds(s * rps + r * 128, 128)
        pltpu.make_async_copy(acc_sh.at[sl], acc1.at[sl], zsem).start()
      for r in range(rps // 128):
        sl = pl.ds(s * rps + r * 128, 128)
        pltpu.make_async_copy(acc_sh.at[sl], acc1.at[sl], zsem).wait()

    if with_deg:
      pltpu.sync_copy(degbuf, degp.at[wid])

  return pl.kernel(body, out_type=tuple(out_type), mesh=mesh,
                   scratch_types=tuple(scratch), compiler_params=_SC_PARAMS)


def _conv2_filtered_kernel(n_pad, feat, n_edges, n_b):
  epw = n_edges // NW
  SP = 2 * n_b
  TRASH = n_b
  R = 5
  assert epw % 16 == 0 and n_b % 128 == 0
  rps = SP // NS
  dps = n_b // NS
  mesh = plsc.VectorSubcoreMesh(core_axis_name="c", subcore_axis_name="s")
  out_type = jax.ShapeDtypeStruct((NC, n_b, feat), jnp.float32)
  scratch = (
      pltpu.VMEM((epw,), jnp.int32),
      pltpu.VMEM((epw,), jnp.int32),
      pltpu.VMEM((n_pad,), jnp.int32),
      pltpu.VMEM((epw + 16,), jnp.int32),
      pltpu.VMEM((epw + 16,), jnp.int32),
      pltpu.VMEM((1, n_b), jnp.int32),
      pltpu.VMEM((R, 16, feat), jnp.float32),
      pltpu.VMEM((64, feat), jnp.float32),
      pltpu.VMEM_SHARED((SP, feat), jnp.float32),
      pltpu.SemaphoreType.DMA,
  ) + (pltpu.SemaphoreType.DMA,) * R \
    + (pltpu.SemaphoreType.DMA,) * R \
    + (pltpu.SemaphoreType.DMA,)

  def body(table, src_r, dst_r, center, bout,
           srcv, dstv, slotmap, fsrc, fslot, cbuf, rows, zbuf, acc_sh,
           esem, *sems):
    gsem = sems[:R]
    ssem = sems[R:2 * R]
    zsem = sems[2 * R]
    c = lax.axis_index("c")
    s = lax.axis_index("s")
    wid = c * NS + s
    wbase = pl.multiple_of(wid * epw, 8)
    z16 = jnp.zeros((16,), jnp.float32)
    zi16 = jnp.zeros((16,), jnp.int32)
    iota16 = lax.broadcasted_iota(jnp.int32, (16,), 0)
    gpr = feat // 16

    pltpu.make_async_copy(src_r.at[pl.ds(wbase, epw)], srcv, esem).start()
    pltpu.make_async_copy(dst_r.at[pl.ds(wbase, epw)], dstv, esem).start()
    pltpu.make_async_copy(center, cbuf.at[0], esem).start()

    def zb(t, carry):
      zbuf[t // gpr, pl.ds((t % gpr) * 16, 16)] = z16
      return carry
    lax.fori_loop(0, 64 * gpr, zb, 0)
    for r in range(rps // 64):
      pltpu.make_async_copy(
          zbuf, acc_sh.at[pl.ds(s * rps + r * 64, 64)], zsem).start()

    trash16 = jnp.full((16,), TRASH, jnp.int32)

    def fill(t, carry):
      slotmap[pl.ds(t * 16, 16)] = trash16
      return carry
    lax.fori_loop(0, n_pad // 16, fill, 0)
    pltpu.make_async_copy(center, cbuf.at[0], esem).wait()

    def scat(g, carry):
      idx = cbuf[0, pl.ds(g * 16, 16)]
      plsc.store_scatter(slotmap, [idx], g * 16 + iota16)
      return carry
    lax.fori_loop(0, n_b // 16, scat, 0)

    pltpu.make_async_copy(src_r.at[pl.ds(wbase, epw)], srcv, esem).wait()
    pltpu.make_async_copy(dst_r.at[pl.ds(wbase, epw)], dstv, esem).wait()
    for r in range(rps // 64):
      pltpu.make_async_copy(
          zbuf, acc_sh.at[pl.ds(s * rps + r * 64, 64)], zsem).wait()
    plsc.subcore_barrier()

    def filt(i, ptr):
      d16 = dstv[pl.ds(i * 16, 16)]
      s16 = srcv[pl.ds(i * 16, 16)]
      sl16 = plsc.load_gather(slotmap, [d16])
      m = sl16 < TRASH
      plsc.store_compressed(fsrc.at[pl.ds(ptr, 16)], s16, mask=m)
      plsc.store_compressed(fslot.at[pl.ds(ptr, 16)], sl16, mask=m)
      cnt = plsc.all_reduce_population_count(m)[0]
      return ptr + cnt
    nf = lax.fori_loop(0, epw // 16, filt, jnp.int32(0))
    fsrc[pl.ds(nf, 16)] = zi16
    fslot[pl.ds(nf, 16)] = trash16
    t2 = (nf + 15) // 16

    def fire_gather(j, u):
      sidx = fsrc[pl.ds(j * 16, 16)]
      pltpu.make_async_copy(table.at[sidx], rows.at[u], gsem[u]).start()

    def wait_gather(u):
      pltpu.make_async_copy(table.at[zi16], rows.at[u], gsem[u]).wait()

    def fire_scatter(j, u):
      didx = fslot[pl.ds(j * 16, 16)]
      pltpu.async_copy(rows.at[u], acc_sh.at[didx], ssem[u], add=True)

    def wait_scatter(u):
      pltpu.make_async_copy(rows.at[u], acc_sh.at[zi16], ssem[u]).wait()

    for jp in range(3):
      @pl.when(jp < t2)
      def _(jp=jp):
        fire_gather(jp, jp)

    def pb(t, carry):
      for u in range(R):
        j = t * R + u
        j2 = j + 3
        w = (u + 3) % R

        @pl.when(jnp.logical_and(j2 < t2, j2 >= R))
        def _(w=w):
          wait_scatter(w)

        @pl.when(j2 < t2)
        def _(j2=j2, w=w):
          fire_gather(j2, w)

        @pl.when(j < t2)
        def _(j=j, u=u):
          wait_gather(u)
          fire_scatter(j, u)
      return carry
    lax.fori_loop(0, (t2 + R - 1) // R, pb, 0)
    for u in range(R):
      @pl.when(u < t2)
      def _(u=u):
        wait_scatter(u)
    plsc.subcore_barrier()

    pltpu.sync_copy(acc_sh.at[pl.ds(s * dps, dps)],
                    bout.at[c].at[pl.ds(s * dps, dps)])

  return pl.kernel(body, out_type=out_type, mesh=mesh, scratch_types=scratch,
                   compiler_params=_SC_PARAMS)


def _center_gather_kernel(n_pad, feat, n_b):
  bpw = n_b // NW
  TRASH = n_b
  assert bpw * NW == n_b and bpw % 16 == 0
  ngr = bpw // 16
  mesh = plsc.VectorSubcoreMesh(core_axis_name="c", subcore_axis_name="s")
  side_out = (jax.ShapeDtypeStruct((n_b, feat), jnp.float32),
              jax.ShapeDtypeStruct((n_b, feat), jnp.float32),
              jax.ShapeDtypeStruct((n_b,), jnp.float32))
  out_type = side_out + side_out
  side_scratch = (
      pltpu.VMEM((1, n_b), jnp.int32),
      pltpu.VMEM((bpw, feat), jnp.float32),
      pltpu.VMEM((bpw, feat), jnp.float32),
      pltpu.VMEM((n_pad,), jnp.float32),
      pltpu.VMEM((bpw,), jnp.float32),
  ) + tuple(pltpu.VMEM((1, 16), jnp.int32) for _ in range(ngr))
  scratch = (pltpu.VMEM((n_pad,), jnp.int32),) + side_scratch + side_scratch \
      + (pltpu.SemaphoreType.DMA, pltpu.SemaphoreType.DMA)

  def body(b0c, b1c, invdc, ctrc, b0t, b1t, invdt, ctrt,
           r0c, r1c, idegcc, r0t, r1t, idegct, *rest):
    slotmap = rest[0]
    nsb = 5 + ngr
    sides = (
        (b0c, b1c, invdc, ctrc, r0c, r1c, idegcc, rest[1:1 + nsb]),
        (b0t, b1t, invdt, ctrt, r0t, r1t, idegct, rest[1 + nsb:1 + 2 * nsb]),
    )
    esem, gsem = rest[1 + 2 * nsb:3 + 2 * nsb]
    c = lax.axis_index("c")
    s = lax.axis_index("s")
    base = (c * NS + s) * bpw
    iota16 = lax.broadcasted_iota(jnp.int32, (16,), 0)
    trash16 = jnp.full((16,), TRASH, jnp.int32)

    for (_, _, invdeg, center, _, _, _, sb) in sides:
      cbuf, _, _, degv, _ = sb[:5]
      pltpu.make_async_copy(center, cbuf.at[0], esem).start()
      pltpu.make_async_copy(invdeg, degv, esem).start()

    for (b0, b1, invdeg, center, r0, r1, idegc, sb) in sides:
      cbuf, rows0, rows1, degv, degc = sb[:5]
      slb = sb[5:]

      def fill(t, carry):
        slotmap[pl.ds(t * 16, 16)] = trash16
        return carry
      lax.fori_loop(0, n_pad // 16, fill, 0)
      pltpu.make_async_copy(center, cbuf.at[0], esem).wait()

      def scat(g, carry):
        idx = cbuf[0, pl.ds(g * 16, 16)]
        plsc.store_scatter(slotmap, [idx], g * 16 + iota16)
        return carry
      lax.fori_loop(0, n_b // 16, scat, 0)

      pltpu.make_async_copy(invdeg, degv, esem).wait()
      for g in range(ngr):
        cidx = cbuf[0, pl.ds(base + g * 16, 16)]
        sl16 = plsc.load_gather(slotmap, [cidx])
        slb[g][0, pl.ds(0, 16)] = sl16
        pltpu.make_async_copy(b0.at[slb[g].at[0]],
                              rows0.at[pl.ds(g * 16, 16)], gsem).start()
        pltpu.make_async_copy(b1.at[slb[g].at[0]],
                              rows1.at[pl.ds(g * 16, 16)], gsem).start()
        degc[pl.ds(g * 16, 16)] = plsc.load_gather(degv, [cidx])

    for (b0, b1, _, _, r0, r1, idegc, sb) in sides:
      _, rows0, rows1, _, degc = sb[:5]
      slb = sb[5:]
      for g in range(ngr):
        pltpu.make_async_copy(b0.at[slb[g].at[0]],
                              rows0.at[pl.ds(g * 16, 16)], gsem).wait()
        pltpu.make_async_copy(b1.at[slb[g].at[0]],
                              rows1.at[pl.ds(g * 16, 16)], gsem).wait()
      pltpu.sync_copy(rows0, r0.at[pl.ds(base, bpw)])
      pltpu.sync_copy(rows1, r1.at[pl.ds(base, bpw)])
      pltpu.sync_copy(degc, idegc.at[pl.ds(base, bpw)])

  return pl.kernel(body, out_type=out_type, mesh=mesh, scratch_types=scratch,
                   compiler_params=_SC_PARAMS)


def _conv_dense(n_pad, d_in, d_out):
  blk = 1280
  grid = (n_pad // blk,)

  def body(a0, a1, degp, w, bb, h, invd):
    deg = jnp.sum(degp[...], axis=1)
    inv = 1.0 / jnp.maximum(deg, 1.0)
    agg = (a0[...] + a1[...]) * inv[:, None]
    h[...] = jnp.maximum(
        jnp.dot(agg, w[...], preferred_element_type=jnp.float32) + bb[...], 0.0)
    invd[...] = inv[:, None]

  return pl.pallas_call(
      body,
      grid=grid,
      in_specs=[
          pl.BlockSpec((blk, d_in), lambda i: (i, 0)),
          pl.BlockSpec((blk, d_in), lambda i: (i, 0)),
          pl.BlockSpec((blk, NW), lambda i: (i, 0)),
          pl.BlockSpec((d_in, d_out), lambda i: (0, 0)),
          pl.BlockSpec((1, d_out), lambda i: (0, 0)),
      ],
      out_specs=[
          pl.BlockSpec((blk, d_out), lambda i: (i, 0)),
          pl.BlockSpec((blk, 1), lambda i: (i, 0)),
      ],
      out_shape=[
          jax.ShapeDtypeStruct((n_pad, d_out), jnp.float32),
          jax.ShapeDtypeStruct((n_pad, 1), jnp.float32),
      ],
  )


def _final_kernel(h, n_b, h2):
  def body(r0c, r1c, idc, r0t, r1t, idt, cw2, cb2, tw2, tb2,
           pw1, pb1, pw2, pb2, zp, zt):
    zc = jnp.dot((r0c[...] + r1c[...]) * idc[...], cw2[...],
                 preferred_element_type=jnp.float32) + cb2[...]
    hid = jnp.maximum(
        jnp.dot(zc, pw1[...], preferred_element_type=jnp.float32) + pb1[...],
        0.0)
    zp[...] = jnp.dot(hid, pw2[...],
                      preferred_element_type=jnp.float32) + pb2[...]
    zt[...] = jnp.dot((r0t[...] + r1t[...]) * idt[...], tw2[...],
                      preferred_element_type=jnp.float32) + tb2[...]

  return pl.pallas_call(
      body,
      out_shape=[jax.ShapeDtypeStruct((n_b, h), jnp.float32),
                 jax.ShapeDtypeStruct((n_b, h), jnp.float32)])


def kernel(x_context, edge_index_context, center_mask_context,
           x_target, edge_index_target, center_mask_target,
           cW1, cb1, cW2, cb2, tW1, tb1, tW2, tb2,
           pW1, pb1, pW2, pb2):
  n, d = x_context.shape
  e = edge_index_context.shape[1]
  nb = center_mask_context.shape[0]
  h = cW1.shape[1]
  h2 = pW1.shape[1]
  n_pad = -(-n // (NS * 128)) * (NS * 128)

  seg_deg = _seg_sum_kernel(n_pad, d, e, True)
  seg2 = _conv2_filtered_kernel(n_pad, h, e, nb)
  conv = _conv_dense(n_pad, d, h)
  gath = _center_gather_kernel(n_pad, h, nb)
  fin = _final_kernel(h, nb, h2)

  def enc(x, ei, ctr, w1, bias1):
    src, dst = ei[0], ei[1]
    a0, a1, degp = seg_deg(x, src, dst)
    hdn, invd = conv(a0, a1, degp.T, w1, bias1.reshape(1, h))
    bout = seg2(hdn, src, dst, ctr)
    return bout, invd.reshape(-1)

  boutc, invdc = enc(x_context, edge_index_context, center_mask_context,
                     cW1, cb1)
  boutt, invdt = enc(x_target, edge_index_target, center_mask_target,
                     tW1, tb1)
  r0c, r1c, idc, r0t, r1t, idt = gath(
      boutc[0], boutc[1], invdc, center_mask_context,
      boutt[0], boutt[1], invdt, center_mask_target)
  zp, zt = fin(r0c, r1c, idc.reshape(nb, 1), r0t, r1t, idt.reshape(nb, 1),
               cW2, cb2.reshape(1, h), tW2, tb2.reshape(1, h),
               pW1, pb1.reshape(1, h2), pW2, pb2.reshape(1, h))
  return (zp, zt)


# speedup vs baseline: 14.6566x; 1.0010x over previous
import jax
import jax.numpy as jnp
from jax import lax
from jax.experimental import pallas as pl
from jax.experimental.pallas import tpu as pltpu
from jax.experimental.pallas import tpu_sc as plsc

NC = 2
NS = 16
NW = NC * NS

_SC_PARAMS = pltpu.CompilerParams(needs_layout_passes=False)


_K = 40
_G = 5


def _seg_sum_kernel(n_pad, feat, n_edges, with_deg):
  K, G = _K, _G
  epw = n_edges // NW
  nchunk = epw // K
  ng = nchunk // G
  nouter = ng // 2
  assert epw * NW == n_edges and K * nchunk == epw and G * ng == nchunk
  assert ng % 2 == 0 and n_pad % (NS * 128) == 0
  rps = n_pad // NS
  mesh = plsc.VectorSubcoreMesh(core_axis_name="c", subcore_axis_name="s")
  out_type = [jax.ShapeDtypeStruct((n_pad, feat), jnp.float32),
              jax.ShapeDtypeStruct((n_pad, feat), jnp.float32)]
  if with_deg:
    out_type.append(jax.ShapeDtypeStruct((NW, n_pad), jnp.float32))
  scratch = [pltpu.VMEM((1, K), jnp.int32)] * (2 * G) \
    + [pltpu.VMEM((1, K), jnp.int32)] * (2 * G) \
    + [pltpu.VMEM((1, K), jnp.int32)] * G \
    + [
      pltpu.VMEM((G, K, feat), jnp.float32),
      pltpu.VMEM((64, feat), jnp.float32),
      pltpu.VMEM_SHARED((n_pad, feat), jnp.float32),
      pltpu.SemaphoreType.DMA,
      pltpu.SemaphoreType.DMA,
  ] + [pltpu.SemaphoreType.DMA] * G \
    + [pltpu.SemaphoreType.DMA] * G \
    + [pltpu.SemaphoreType.DMA]
  if with_deg:
    scratch.append(pltpu.VMEM((n_pad,), jnp.float32))

  def body(table, src_r, dst_r, acc0, acc1, *rest):
    if with_deg:
      degp = rest[0]
      rest = rest[1:]
    srcb = rest[:2 * G]
    dstb = rest[2 * G:4 * G]
    sdst = rest[4 * G:5 * G]
    rows, zbuf, acc_sh = rest[5 * G:5 * G + 3]
    off0 = 5 * G + 3
    isem = rest[off0:off0 + 2]
    gsem = rest[off0 + 2:off0 + 2 + G]
    ssem = rest[off0 + 2 + G:off0 + 2 + 2 * G]
    zsem = rest[off0 + 2 + 2 * G]
    degbuf = rest[off0 + 3 + 2 * G] if with_deg else None
    c = lax.axis_index("c")
    s = lax.axis_index("s")
    wid = c * NS + s
    wbase = wid * epw
    z16 = jnp.zeros((16,), jnp.float32)
    ones = jnp.ones((16,), jnp.float32)
    tailmask = lax.broadcasted_iota(jnp.int32, (16,), 0) >= 8
    gpr = feat // 16

    def zb(t, carry):
      zbuf[t // gpr, pl.ds((t % gpr) * 16, 16)] = z16
      return carry
    lax.fori_loop(0, 64 * gpr, zb, 0)
    for r in range(rps // 64):
      pltpu.make_async_copy(
          zbuf, acc_sh.at[pl.ds(s * rps + r * 64, 64)], zsem).start()
    if with_deg:
      def zd(t, carry):
        degbuf[pl.ds(t * 16, 16)] = z16
        return carry
      lax.fori_loop(0, n_pad // 16, zd, 0)
    for r in range(rps // 64):
      pltpu.make_async_copy(
          zbuf, acc_sh.at[pl.ds(s * rps + r * 64, 64)], zsem).wait()
    plsc.subcore_barrier()

    def fire_idx_group(gi, q):
      off = pl.multiple_of(wbase + gi * (G * K), 8)
      for uu in range(G):
        pltpu.make_async_copy(src_r.at[pl.ds(off + uu * K, K)],
                              srcb[q * G + uu].at[0], isem[q]).start()
        pltpu.make_async_copy(dst_r.at[pl.ds(off + uu * K, K)],
                              dstb[q * G + uu].at[0], isem[q]).start()

    def wait_idx_group(q):
      off = pl.multiple_of(wbase, 8)
      for uu in range(G):
        pltpu.make_async_copy(src_r.at[pl.ds(off + uu * K, K)],
                              srcb[q * G + uu].at[0], isem[q]).wait()
        pltpu.make_async_copy(dst_r.at[pl.ds(off + uu * K, K)],
                              dstb[q * G + uu].at[0], isem[q]).wait()

    def fire_gather(q, u):
      pltpu.make_async_copy(table.at[srcb[q * G + u].at[0]],
                            rows.at[u], gsem[u]).start()

    def wait_gather(u):
      pltpu.make_async_copy(table.at[srcb[u].at[0]],
                            rows.at[u], gsem[u]).wait()

    def fire_scatter(u):
      pltpu.async_copy(rows.at[u], acc_sh.at[sdst[u].at[0]],
                       ssem[u], add=True)

    def wait_scatter(u):
      pltpu.make_async_copy(rows.at[u], acc_sh.at[sdst[u].at[0]],
                            ssem[u]).wait()

    def drain_chunk(q, u):
      wait_gather(u)
      d = dstb[q * G + u]
      for o in (0, 16, 24):
        sdst[u][0, pl.ds(o, 16)] = d[0, pl.ds(o, 16)]
      fire_scatter(u)
      if with_deg:
        d = dstb[q * G + u]
        plsc.addupdate_scatter(degbuf, [d[0, pl.ds(0, 16)]], ones)
        plsc.addupdate_scatter(degbuf, [d[0, pl.ds(16, 16)]], ones)
        plsc.addupdate_scatter(degbuf, [d[0, pl.ds(24, 16)]], ones,
                               mask=tailmask)

    offp = pl.multiple_of(wbase, 8)
    for uu in range(G):
      pltpu.sync_copy(src_r.at[pl.ds(offp + uu * K, K)], srcb[uu].at[0])
      pltpu.sync_copy(dst_r.at[pl.ds(offp + uu * K, K)], dstb[uu].at[0])
    fire_idx_group(1, 1)
    fire_gather(0, 0)
    fire_gather(0, 1)

    def outer(gg, carry):
      for pp in (0, 1):
        for u in range(G):
          if u < 3:
            w = u + 2
            if pp == 0:
              @pl.when(gg > 0)
              def _(w=w):
                wait_scatter(w)
            else:
              wait_scatter(w)
            fire_gather(pp, w)
          else:
            w = u - 3
            if pp == 0:
              if u == 3:
                wait_idx_group(1)
              wait_scatter(w)
              fire_gather(1, w)
            else:
              @pl.when(gg < nouter - 1)
              def _(w=w, u=u):
                if u == 3:
                  wait_idx_group(0)
                wait_scatter(w)
                fire_gather(0, w)
          drain_chunk(pp, u)
          if u == 4:
            @pl.when(gg < nouter - 1)
            def _(pp=pp):
              fire_idx_group(gg * 2 + 2 + pp, pp)
      return carry

    lax.fori_loop(0, nouter, outer, 0)
    for u in range(G):
      wait_scatter(u)
    plsc.subcore_barrier()

    @pl.when(c == 0)
    def _():
      for r in range(rps // 128):
        sl = pl.ds(s * rps + r * 128, 128)
        pltpu.make_async_copy(acc_sh.at[sl], acc0.at[sl], zsem).start()
      for r in range(rps // 128):
        sl = pl.ds(s * rps + r * 128, 128)
        pltpu.make_async_copy(acc_sh.at[sl], acc0.at[sl], zsem).wait()

    @pl.when(c == 1)
    def _():
      for r in range(rps // 128):
        sl = pl.ds(s * rps + r * 128, 128)
        pltpu.make_async_copy(acc_sh.at[sl], acc1.at[sl], zsem).start()
      for r in range(rps // 128):
        sl = pl.ds(s * rps + r * 128, 128)
        pltpu.make_async_copy(acc_sh.at[sl], acc1.at[sl], zsem).wait()

    if with_deg:
      pltpu.sync_copy(degbuf, degp.at[wid])

  return pl.kernel(body, out_type=tuple(out_type), mesh=mesh,
                   scratch_types=tuple(scratch), compiler_params=_SC_PARAMS)


def _conv2_filtered_kernel(n_pad, feat, n_edges, n_b):
  epw = n_edges // NW
  SP = 2 * n_b
  TRASH = n_b
  R = 5
  assert epw % 16 == 0 and n_b % 128 == 0
  rps = SP // NS
  dps = n_b // NS
  mesh = plsc.VectorSubcoreMesh(core_axis_name="c", subcore_axis_name="s")
  out_type = jax.ShapeDtypeStruct((NC, n_b, feat), jnp.float32)
  scratch = (
      pltpu.VMEM((epw,), jnp.int32),
      pltpu.VMEM((epw,), jnp.int32),
      pltpu.VMEM((n_pad,), jnp.int32),
      pltpu.VMEM((epw + 16,), jnp.int32),
      pltpu.VMEM((epw + 16,), jnp.int32),
      pltpu.VMEM((1, n_b), jnp.int32),
      pltpu.VMEM((R, 16, feat), jnp.float32),
      pltpu.VMEM((64, feat), jnp.float32),
      pltpu.VMEM_SHARED((SP, feat), jnp.float32),
      pltpu.SemaphoreType.DMA,
  ) + (pltpu.SemaphoreType.DMA,) * R \
    + (pltpu.SemaphoreType.DMA,) * R \
    + (pltpu.SemaphoreType.DMA,)

  def body(table, src_r, dst_r, center, bout,
           srcv, dstv, slotmap, fsrc, fslot, cbuf, rows, zbuf, acc_sh,
           esem, *sems):
    gsem = sems[:R]
    ssem = sems[R:2 * R]
    zsem = sems[2 * R]
    c = lax.axis_index("c")
    s = lax.axis_index("s")
    wid = c * NS + s
    wbase = pl.multiple_of(wid * epw, 8)
    z16 = jnp.zeros((16,), jnp.float32)
    zi16 = jnp.zeros((16,), jnp.int32)
    iota16 = lax.broadcasted_iota(jnp.int32, (16,), 0)
    gpr = feat // 16

    pltpu.make_async_copy(src_r.at[pl.ds(wbase, epw)], srcv, esem).start()
    pltpu.make_async_copy(dst_r.at[pl.ds(wbase, epw)], dstv, esem).start()
    pltpu.make_async_copy(center, cbuf.at[0], esem).start()

    def zb(t, carry):
      zbuf[t // gpr, pl.ds((t % gpr) * 16, 16)] = z16
      return carry
    lax.fori_loop(0, 64 * gpr, zb, 0)
    for r in range(rps // 64):
      pltpu.make_async_copy(
          zbuf, acc_sh.at[pl.ds(s * rps + r * 64, 64)], zsem).start()

    trash16 = jnp.full((16,), TRASH, jnp.int32)

    def fill(t, carry):
      slotmap[pl.ds(t * 16, 16)] = trash16
      return carry
    lax.fori_loop(0, n_pad // 16, fill, 0)
    pltpu.make_async_copy(center, cbuf.at[0], esem).wait()

    def scat(g, carry):
      idx = cbuf[0, pl.ds(g * 16, 16)]
      plsc.store_scatter(slotmap, [idx], g * 16 + iota16)
      return carry
    lax.fori_loop(0, n_b // 16, scat, 0)

    pltpu.make_async_copy(src_r.at[pl.ds(wbase, epw)], srcv, esem).wait()
    pltpu.make_async_copy(dst_r.at[pl.ds(wbase, epw)], dstv, esem).wait()
    for r in range(rps // 64):
      pltpu.make_async_copy(
          zbuf, acc_sh.at[pl.ds(s * rps + r * 64, 64)], zsem).wait()
    plsc.subcore_barrier()

    def filt(i, ptr):
      d16 = dstv[pl.ds(i * 16, 16)]
      s16 = srcv[pl.ds(i * 16, 16)]
      sl16 = plsc.load_gather(slotmap, [d16])
      m = sl16 < TRASH
      plsc.store_compressed(fsrc.at[pl.ds(ptr, 16)], s16, mask=m)
      plsc.store_compressed(fslot.at[pl.ds(ptr, 16)], sl16, mask=m)
      cnt = plsc.all_reduce_population_count(m)[0]
      return ptr + cnt
    nf = lax.fori_loop(0, epw // 16, filt, jnp.int32(0))
    fsrc[pl.ds(nf, 16)] = zi16
    fslot[pl.ds(nf, 16)] = trash16
    t2 = (nf + 15) // 16

    def fire_gather(j, u):
      sidx = fsrc[pl.ds(j * 16, 16)]
      pltpu.make_async_copy(table.at[sidx], rows.at[u], gsem[u]).start()

    def wait_gather(u):
      pltpu.make_async_copy(table.at[zi16], rows.at[u], gsem[u]).wait()

    def fire_scatter(j, u):
      didx = fslot[pl.ds(j * 16, 16)]
      pltpu.async_copy(rows.at[u], acc_sh.at[didx], ssem[u], add=True)

    def wait_scatter(u):
      pltpu.make_async_copy(rows.at[u], acc_sh.at[zi16], ssem[u]).wait()

    for jp in range(3):
      @pl.when(jp < t2)
      def _(jp=jp):
        fire_gather(jp, jp)

    def pb(t, carry):
      for u in range(R):
        j = t * R + u
        j2 = j + 3
        w = (u + 3) % R

        @pl.when(jnp.logical_and(j2 < t2, j2 >= R))
        def _(w=w):
          wait_scatter(w)

        @pl.when(j2 < t2)
        def _(j2=j2, w=w):
          fire_gather(j2, w)

        @pl.when(j < t2)
        def _(j=j, u=u):
          wait_gather(u)
          fire_scatter(j, u)
      return carry
    lax.fori_loop(0, (t2 + R - 1) // R, pb, 0)
    for u in range(R):
      @pl.when(u < t2)
      def _(u=u):
        wait_scatter(u)
    plsc.subcore_barrier()

    pltpu.sync_copy(acc_sh.at[pl.ds(s * dps, dps)],
                    bout.at[c].at[pl.ds(s * dps, dps)])

  return pl.kernel(body, out_type=out_type, mesh=mesh, scratch_types=scratch,
                   compiler_params=_SC_PARAMS)


def _center_gather_kernel(n_pad, feat, n_b):
  bpw = n_b // NW
  TRASH = n_b
  assert bpw * NW == n_b and bpw % 16 == 0
  ngr = bpw // 16
  mesh = plsc.VectorSubcoreMesh(core_axis_name="c", subcore_axis_name="s")
  side_out = (jax.ShapeDtypeStruct((n_b, feat), jnp.float32),
              jax.ShapeDtypeStruct((n_b, feat), jnp.float32),
              jax.ShapeDtypeStruct((n_b,), jnp.float32))
  out_type = side_out + side_out
  side_scratch = (
      pltpu.VMEM((1, n_b), jnp.int32),
      pltpu.VMEM((bpw, feat), jnp.float32),
      pltpu.VMEM((bpw, feat), jnp.float32),
      pltpu.VMEM((n_pad,), jnp.float32),
      pltpu.VMEM((bpw,), jnp.float32),
  ) + tuple(pltpu.VMEM((1, 16), jnp.int32) for _ in range(ngr))
  scratch = (pltpu.VMEM((n_pad,), jnp.int32),) + side_scratch + side_scratch \
      + (pltpu.SemaphoreType.DMA, pltpu.SemaphoreType.DMA)

  def body(b0c, b1c, invdc, ctrc, b0t, b1t, invdt, ctrt,
           r0c, r1c, idegcc, r0t, r1t, idegct, *rest):
    slotmap = rest[0]
    nsb = 5 + ngr
    sides = (
        (b0c, b1c, invdc, ctrc, r0c, r1c, idegcc, rest[1:1 + nsb]),
        (b0t, b1t, invdt, ctrt, r0t, r1t, idegct, rest[1 + nsb:1 + 2 * nsb]),
    )
    esem, gsem = rest[1 + 2 * nsb:3 + 2 * nsb]
    c = lax.axis_index("c")
    s = lax.axis_index("s")
    base = (c * NS + s) * bpw
    iota16 = lax.broadcasted_iota(jnp.int32, (16,), 0)
    trash16 = jnp.full((16,), TRASH, jnp.int32)

    for (_, _, invdeg, center, _, _, _, sb) in sides:
      cbuf, _, _, degv, _ = sb[:5]
      pltpu.make_async_copy(center, cbuf.at[0], esem).start()
      pltpu.make_async_copy(invdeg, degv, esem).start()

    for (b0, b1, invdeg, center, r0, r1, idegc, sb) in sides:
      cbuf, rows0, rows1, degv, degc = sb[:5]
      slb = sb[5:]

      def fill(t, carry):
        slotmap[pl.ds(t * 16, 16)] = trash16
        return carry
      lax.fori_loop(0, n_pad // 16, fill, 0)
      pltpu.make_async_copy(center, cbuf.at[0], esem).wait()

      def scat(g, carry):
        idx = cbuf[0, pl.ds(g * 16, 16)]
        plsc.store_scatter(slotmap, [idx], g * 16 + iota16)
        return carry
      lax.fori_loop(0, n_b // 16, scat, 0)

      pltpu.make_async_copy(invdeg, degv, esem).wait()
      for g in range(ngr):
        cidx = cbuf[0, pl.ds(base + g * 16, 16)]
        sl16 = plsc.load_gather(slotmap, [cidx])
        slb[g][0, pl.ds(0, 16)] = sl16
        pltpu.make_async_copy(b0.at[slb[g].at[0]],
                              rows0.at[pl.ds(g * 16, 16)], gsem).start()
        pltpu.make_async_copy(b1.at[slb[g].at[0]],
                              rows1.at[pl.ds(g * 16, 16)], gsem).start()
        degc[pl.ds(g * 16, 16)] = plsc.load_gather(degv, [cidx])

    for (b0, b1, _, _, r0, r1, idegc, sb) in sides:
      _, rows0, rows1, _, degc = sb[:5]
      slb = sb[5:]
      for g in range(ngr):
        pltpu.make_async_copy(b0.at[slb[g].at[0]],
                              rows0.at[pl.ds(g * 16, 16)], gsem).wait()
        pltpu.make_async_copy(b1.at[slb[g].at[0]],
                              rows1.at[pl.ds(g * 16, 16)], gsem).wait()
      pltpu.sync_copy(rows0, r0.at[pl.ds(base, bpw)])
      pltpu.sync_copy(rows1, r1.at[pl.ds(base, bpw)])
      pltpu.sync_copy(degc, idegc.at[pl.ds(base, bpw)])

  return pl.kernel(body, out_type=out_type, mesh=mesh, scratch_types=scratch,
                   compiler_params=_SC_PARAMS)


def _conv_dense(n_pad, d_in, d_out):
  blk = 1280
  grid = (n_pad // blk,)

  def body(a0, a1, degp, w, bb, h, invd):
    deg = jnp.sum(degp[...], axis=1)
    inv = 1.0 / jnp.maximum(deg, 1.0)
    agg = (a0[...] + a1[...]) * inv[:, None]
    h[...] = jnp.maximum(
        jnp.dot(agg, w[...], preferred_element_type=jnp.float32) + bb[...], 0.0)
    invd[...] = inv[:, None]

  return pl.pallas_call(
      body,
      grid=grid,
      in_specs=[
          pl.BlockSpec((blk, d_in), lambda i: (i, 0)),
          pl.BlockSpec((blk, d_in), lambda i: (i, 0)),
          pl.BlockSpec((blk, NW), lambda i: (i, 0)),
          pl.BlockSpec((d_in, d_out), lambda i: (0, 0)),
          pl.BlockSpec((1, d_out), lambda i: (0, 0)),
      ],
      out_specs=[
          pl.BlockSpec((blk, d_out), lambda i: (i, 0)),
          pl.BlockSpec((blk, 1), lambda i: (i, 0)),
      ],
      out_shape=[
          jax.ShapeDtypeStruct((n_pad, d_out), jnp.float32),
          jax.ShapeDtypeStruct((n_pad, 1), jnp.float32),
      ],
  )


def _final_kernel(h, n_b, h2):
  def body(r0c, r1c, idc, r0t, r1t, idt, cw2, cb2, tw2, tb2,
           pw1, pb1, pw2, pb2, zp, zt):
    zc = jnp.dot((r0c[...] + r1c[...]) * idc[...], cw2[...],
                 preferred_element_type=jnp.float32) + cb2[...]
    hid = jnp.maximum(
        jnp.dot(zc, pw1[...], preferred_element_type=jnp.float32) + pb1[...],
        0.0)
    zp[...] = jnp.dot(hid, pw2[...],
                      preferred_element_type=jnp.float32) + pb2[...]
    zt[...] = jnp.dot((r0t[...] + r1t[...]) * idt[...], tw2[...],
                      preferred_element_type=jnp.float32) + tb2[...]

  return pl.pallas_call(
      body,
      out_shape=[jax.ShapeDtypeStruct((n_b, h), jnp.float32),
                 jax.ShapeDtypeStruct((n_b, h), jnp.float32)])


def kernel(x_context, edge_index_context, center_mask_context,
           x_target, edge_index_target, center_mask_target,
           cW1, cb1, cW2, cb2, tW1, tb1, tW2, tb2,
           pW1, pb1, pW2, pb2):
  n, d = x_context.shape
  e = edge_index_context.shape[1]
  nb = center_mask_context.shape[0]
  h = cW1.shape[1]
  h2 = pW1.shape[1]
  n_pad = -(-n // (NS * 128)) * (NS * 128)

  seg_deg = _seg_sum_kernel(n_pad, d, e, True)
  seg2 = _conv2_filtered_kernel(n_pad, h, e, nb)
  conv = _conv_dense(n_pad, d, h)
  gath = _center_gather_kernel(n_pad, h, nb)
  fin = _final_kernel(h, nb, h2)

  def enc(x, ei, ctr, w1, bias1):
    src, dst = ei[0], ei[1]
    a0, a1, degp = seg_deg(x, src, dst)
    hdn, invd = conv(a0, a1, degp.T, w1, bias1.reshape(1, h))
    bout = seg2(hdn, src, dst, ctr)
    return bout, invd.reshape(-1)

  boutc, invdc = enc(x_context, edge_index_context, center_mask_context,
                     cW1, cb1)
  boutt, invdt = enc(x_target, edge_index_target, center_mask_target,
                     tW1, tb1)
  r0c, r1c, idc, r0t, r1t, idt = gath(
      boutc[0], boutc[1], invdc, center_mask_context,
      boutt[0], boutt[1], invdt, center_mask_target)
  zp, zt = fin(r0c, r1c, idc.reshape(nb, 1), r0t, r1t, idt.reshape(nb, 1),
               cW2, cb2.reshape(1, h), tW2, tb2.reshape(1, h),
               pW1, pb1.reshape(1, h2), pW2, pb2.reshape(1, h))
  return (zp, zt)


# speedup vs baseline: 15.5944x; 1.0640x over previous
import jax
import jax.numpy as jnp
from jax import lax
from jax.experimental import pallas as pl
from jax.experimental.pallas import tpu as pltpu
from jax.experimental.pallas import tpu_sc as plsc

NC = 2
NS = 16
NW = NC * NS

_SC_PARAMS = pltpu.CompilerParams(needs_layout_passes=False)


_K = 40
_G = 5


def _seg_sum_kernel(n_pad, feat, n_edges, with_deg):
  K, G = _K, _G
  epw = n_edges // NW
  nchunk = epw // K
  ng = nchunk // G
  nouter = ng // 2
  assert epw * NW == n_edges and K * nchunk == epw and G * ng == nchunk
  assert ng % 2 == 0 and n_pad % (NS * 128) == 0
  rps = n_pad // NS
  mesh = plsc.VectorSubcoreMesh(core_axis_name="c", subcore_axis_name="s")
  out_type = [jax.ShapeDtypeStruct((n_pad, feat), jnp.float32),
              jax.ShapeDtypeStruct((n_pad, feat), jnp.float32)]
  if with_deg:
    out_type.append(jax.ShapeDtypeStruct((NW, n_pad), jnp.float32))
  scratch = [pltpu.VMEM((1, K), jnp.int32)] * (2 * G) \
    + [pltpu.VMEM((1, K), jnp.int32)] * (2 * G) \
    + [pltpu.VMEM((1, K), jnp.int32)] * G \
    + [
      pltpu.VMEM((G, K, feat), jnp.float32),
      pltpu.VMEM((64, feat), jnp.float32),
      pltpu.VMEM_SHARED((n_pad, feat), jnp.float32),
      pltpu.SemaphoreType.DMA,
      pltpu.SemaphoreType.DMA,
  ] + [pltpu.SemaphoreType.DMA] * G \
    + [pltpu.SemaphoreType.DMA] * G \
    + [pltpu.SemaphoreType.DMA]
  if with_deg:
    scratch.append(pltpu.VMEM((n_pad,), jnp.float32))

  def body(table, src_r, dst_r, acc0, acc1, *rest):
    if with_deg:
      degp = rest[0]
      rest = rest[1:]
    srcb = rest[:2 * G]
    dstb = rest[2 * G:4 * G]
    sdst = rest[4 * G:5 * G]
    rows, zbuf, acc_sh = rest[5 * G:5 * G + 3]
    off0 = 5 * G + 3
    isem = rest[off0:off0 + 2]
    gsem = rest[off0 + 2:off0 + 2 + G]
    ssem = rest[off0 + 2 + G:off0 + 2 + 2 * G]
    zsem = rest[off0 + 2 + 2 * G]
    degbuf = rest[off0 + 3 + 2 * G] if with_deg else None
    c = lax.axis_index("c")
    s = lax.axis_index("s")
    wid = c * NS + s
    wbase = wid * epw
    z16 = jnp.zeros((16,), jnp.float32)
    ones = jnp.ones((16,), jnp.float32)
    tailmask = lax.broadcasted_iota(jnp.int32, (16,), 0) >= 8
    gpr = feat // 16

    def zb(t, carry):
      zbuf[t // gpr, pl.ds((t % gpr) * 16, 16)] = z16
      return carry
    lax.fori_loop(0, 64 * gpr, zb, 0)
    for r in range(rps // 64):
      pltpu.make_async_copy(
          zbuf, acc_sh.at[pl.ds(s * rps + r * 64, 64)], zsem).start()
    if with_deg:
      def zd(t, carry):
        degbuf[pl.ds(t * 16, 16)] = z16
        return carry
      lax.fori_loop(0, n_pad // 16, zd, 0)
    for r in range(rps // 64):
      pltpu.make_async_copy(
          zbuf, acc_sh.at[pl.ds(s * rps + r * 64, 64)], zsem).wait()
    plsc.subcore_barrier()

    def fire_idx_group(gi, q):
      off = pl.multiple_of(wbase + gi * (G * K), 8)
      for uu in range(G):
        pltpu.make_async_copy(src_r.at[pl.ds(off + uu * K, K)],
                              srcb[q * G + uu].at[0], isem[q]).start()
        pltpu.make_async_copy(dst_r.at[pl.ds(off + uu * K, K)],
                              dstb[q * G + uu].at[0], isem[q]).start()

    def wait_idx_group(q):
      off = pl.multiple_of(wbase, 8)
      for uu in range(G):
        pltpu.make_async_copy(src_r.at[pl.ds(off + uu * K, K)],
                              srcb[q * G + uu].at[0], isem[q]).wait()
        pltpu.make_async_copy(dst_r.at[pl.ds(off + uu * K, K)],
                              dstb[q * G + uu].at[0], isem[q]).wait()

    def fire_gather(q, u):
      pltpu.make_async_copy(table.at[srcb[q * G + u].at[0]],
                            rows.at[u], gsem[u]).start()

    def wait_gather(u):
      pltpu.make_async_copy(table.at[srcb[u].at[0]],
                            rows.at[u], gsem[u]).wait()

    def fire_scatter(u):
      pltpu.async_copy(rows.at[u], acc_sh.at[sdst[u].at[0]],
                       ssem[u], add=True)

    def wait_scatter(u):
      pltpu.make_async_copy(rows.at[u], acc_sh.at[sdst[u].at[0]],
                            ssem[u]).wait()

    def drain_chunk(q, u):
      wait_gather(u)
      d = dstb[q * G + u]
      for o in (0, 16, 24):
        sdst[u][0, pl.ds(o, 16)] = d[0, pl.ds(o, 16)]
      fire_scatter(u)
      if with_deg:
        d = dstb[q * G + u]
        plsc.addupdate_scatter(degbuf, [d[0, pl.ds(0, 16)]], ones)
        plsc.addupdate_scatter(degbuf, [d[0, pl.ds(16, 16)]], ones)
        plsc.addupdate_scatter(degbuf, [d[0, pl.ds(24, 16)]], ones,
                               mask=tailmask)

    offp = pl.multiple_of(wbase, 8)
    for uu in range(G):
      pltpu.sync_copy(src_r.at[pl.ds(offp + uu * K, K)], srcb[uu].at[0])
      pltpu.sync_copy(dst_r.at[pl.ds(offp + uu * K, K)], dstb[uu].at[0])
    fire_idx_group(1, 1)
    fire_gather(0, 0)
    fire_gather(0, 1)
    fire_gather(0, 2)

    def outer(gg, carry):
      for pp in (0, 1):
        for u in range(G):
          if u < 2:
            w = u + 3
            if pp == 0:
              @pl.when(gg > 0)
              def _(w=w):
                wait_scatter(w)
            else:
              wait_scatter(w)
            fire_gather(pp, w)
          else:
            w = u - 2
            if pp == 0:
              if u == 2:
                wait_idx_group(1)
              wait_scatter(w)
              fire_gather(1, w)
            else:
              @pl.when(gg < nouter - 1)
              def _(w=w, u=u):
                if u == 2:
                  wait_idx_group(0)
                wait_scatter(w)
                fire_gather(0, w)
          drain_chunk(pp, u)
          if u == 4:
            @pl.when(gg < nouter - 1)
            def _(pp=pp):
              fire_idx_group(gg * 2 + 2 + pp, pp)
      return carry

    lax.fori_loop(0, nouter, outer, 0)
    for u in range(G):
      wait_scatter(u)
    plsc.subcore_barrier()

    @pl.when(c == 0)
    def _():
      for r in range(rps // 128):
        sl = pl.ds(s * rps + r * 128, 128)
        pltpu.make_async_copy(acc_sh.at[sl], acc0.at[sl], zsem).start()
      for r in range(rps // 128):
        sl = pl.ds(s * rps + r * 128, 128)
        pltpu.make_async_copy(acc_sh.at[sl], acc0.at[sl], zsem).wait()

    @pl.when(c == 1)
    def _():
      for r in range(rps // 128):
        sl = pl.ds(s * rps + r * 128, 128)
        pltpu.make_async_copy(acc_sh.at[sl], acc1.at[sl], zsem).start()
      for r in range(rps // 128):
        sl = pl.ds(s * rps + r * 128, 128)
        pltpu.make_async_copy(acc_sh.at[sl], acc1.at[sl], zsem).wait()

    if with_deg:
      pltpu.sync_copy(degbuf, degp.at[wid])

  return pl.kernel(body, out_type=tuple(out_type), mesh=mesh,
                   scratch_types=tuple(scratch), compiler_params=_SC_PARAMS)


def _conv2_filtered_kernel(n_pad, feat, n_edges, n_b):
  epw = n_edges // NW
  SP = 2 * n_b
  TRASH = n_b
  R = 5
  assert epw % 16 == 0 and n_b % 128 == 0
  rps = SP // NS
  dps = n_b // NS
  mesh = plsc.VectorSubcoreMesh(core_axis_name="c", subcore_axis_name="s")
  out_type = jax.ShapeDtypeStruct((NC, n_b, feat), jnp.float32)
  scratch = (
      pltpu.VMEM((epw,), jnp.int32),
      pltpu.VMEM((epw,), jnp.int32),
      pltpu.VMEM((n_pad,), jnp.int32),
      pltpu.VMEM((epw + 16,), jnp.int32),
      pltpu.VMEM((epw + 16,), jnp.int32),
      pltpu.VMEM((1, n_b), jnp.int32),
      pltpu.VMEM((R, 16, feat), jnp.float32),
      pltpu.VMEM((64, feat), jnp.float32),
      pltpu.VMEM_SHARED((SP, feat), jnp.float32),
      pltpu.SemaphoreType.DMA,
  ) + (pltpu.SemaphoreType.DMA,) * R \
    + (pltpu.SemaphoreType.DMA,) * R \
    + (pltpu.SemaphoreType.DMA,)

  def body(table, src_r, dst_r, center, bout,
           srcv, dstv, slotmap, fsrc, fslot, cbuf, rows, zbuf, acc_sh,
           esem, *sems):
    gsem = sems[:R]
    ssem = sems[R:2 * R]
    zsem = sems[2 * R]
    c = lax.axis_index("c")
    s = lax.axis_index("s")
    wid = c * NS + s
    wbase = pl.multiple_of(wid * epw, 8)
    z16 = jnp.zeros((16,), jnp.float32)
    zi16 = jnp.zeros((16,), jnp.int32)
    iota16 = lax.broadcasted_iota(jnp.int32, (16,), 0)
    gpr = feat // 16

    pltpu.make_async_copy(src_r.at[pl.ds(wbase, epw)], srcv, esem).start()
    pltpu.make_async_copy(dst_r.at[pl.ds(wbase, epw)], dstv, esem).start()
    pltpu.make_async_copy(center, cbuf.at[0], esem).start()

    def zb(t, carry):
      zbuf[t // gpr, pl.ds((t % gpr) * 16, 16)] = z16
      return carry
    lax.fori_loop(0, 64 * gpr, zb, 0)
    for r in range(rps // 64):
      pltpu.make_async_copy(
          zbuf, acc_sh.at[pl.ds(s * rps + r * 64, 64)], zsem).start()

    trash16 = jnp.full((16,), TRASH, jnp.int32)

    def fill(t, carry):
      slotmap[pl.ds(t * 16, 16)] = trash16
      return carry
    lax.fori_loop(0, n_pad // 16, fill, 0)
    pltpu.make_async_copy(center, cbuf.at[0], esem).wait()

    def scat(g, carry):
      idx = cbuf[0, pl.ds(g * 16, 16)]
      plsc.store_scatter(slotmap, [idx], g * 16 + iota16)
      return carry
    lax.fori_loop(0, n_b // 16, scat, 0)

    pltpu.make_async_copy(src_r.at[pl.ds(wbase, epw)], srcv, esem).wait()
    pltpu.make_async_copy(dst_r.at[pl.ds(wbase, epw)], dstv, esem).wait()
    for r in range(rps // 64):
      pltpu.make_async_copy(
          zbuf, acc_sh.at[pl.ds(s * rps + r * 64, 64)], zsem).wait()
    plsc.subcore_barrier()

    def filt(i, ptr):
      d16 = dstv[pl.ds(i * 16, 16)]
      s16 = srcv[pl.ds(i * 16, 16)]
      sl16 = plsc.load_gather(slotmap, [d16])
      m = sl16 < TRASH
      plsc.store_compressed(fsrc.at[pl.ds(ptr, 16)], s16, mask=m)
      plsc.store_compressed(fslot.at[pl.ds(ptr, 16)], sl16, mask=m)
      cnt = plsc.all_reduce_population_count(m)[0]
      return ptr + cnt
    nf = lax.fori_loop(0, epw // 16, filt, jnp.int32(0))
    fsrc[pl.ds(nf, 16)] = zi16
    fslot[pl.ds(nf, 16)] = trash16
    t2 = (nf + 15) // 16

    def fire_gather(j, u):
      sidx = fsrc[pl.ds(j * 16, 16)]
      pltpu.make_async_copy(table.at[sidx], rows.at[u], gsem[u]).start()

    def wait_gather(u):
      pltpu.make_async_copy(table.at[zi16], rows.at[u], gsem[u]).wait()

    def fire_scatter(j, u):
      didx = fslot[pl.ds(j * 16, 16)]
      pltpu.async_copy(rows.at[u], acc_sh.at[didx], ssem[u], add=True)

    def wait_scatter(u):
      pltpu.make_async_copy(rows.at[u], acc_sh.at[zi16], ssem[u]).wait()

    for jp in range(3):
      @pl.when(jp < t2)
      def _(jp=jp):
        fire_gather(jp, jp)

    def pb(t, carry):
      for u in range(R):
        j = t * R + u
        j2 = j + 3
        w = (u + 3) % R

        @pl.when(jnp.logical_and(j2 < t2, j2 >= R))
        def _(w=w):
          wait_scatter(w)

        @pl.when(j2 < t2)
        def _(j2=j2, w=w):
          fire_gather(j2, w)

        @pl.when(j < t2)
        def _(j=j, u=u):
          wait_gather(u)
          fire_scatter(j, u)
      return carry
    lax.fori_loop(0, (t2 + R - 1) // R, pb, 0)
    for u in range(R):
      @pl.when(u < t2)
      def _(u=u):
        wait_scatter(u)
    plsc.subcore_barrier()

    pltpu.sync_copy(acc_sh.at[pl.ds(s * dps, dps)],
                    bout.at[c].at[pl.ds(s * dps, dps)])

  return pl.kernel(body, out_type=out_type, mesh=mesh, scratch_types=scratch,
                   compiler_params=_SC_PARAMS)


def _center_gather_kernel(n_pad, feat, n_b):
  bpw = n_b // NW
  TRASH = n_b
  assert bpw * NW == n_b and bpw % 16 == 0
  ngr = bpw // 16
  mesh = plsc.VectorSubcoreMesh(core_axis_name="c", subcore_axis_name="s")
  side_out = (jax.ShapeDtypeStruct((n_b, feat), jnp.float32),
              jax.ShapeDtypeStruct((n_b, feat), jnp.float32),
              jax.ShapeDtypeStruct((n_b,), jnp.float32))
  out_type = side_out + side_out
  side_scratch = (
      pltpu.VMEM((1, n_b), jnp.int32),
      pltpu.VMEM((bpw, feat), jnp.float32),
      pltpu.VMEM((bpw, feat), jnp.float32),
      pltpu.VMEM((n_pad,), jnp.float32),
      pltpu.VMEM((bpw,), jnp.float32),
  ) + tuple(pltpu.VMEM((1, 16), jnp.int32) for _ in range(ngr))
  scratch = (pltpu.VMEM((n_pad,), jnp.int32),) + side_scratch + side_scratch \
      + (pltpu.SemaphoreType.DMA, pltpu.SemaphoreType.DMA)

  def body(b0c, b1c, invdc, ctrc, b0t, b1t, invdt, ctrt,
           r0c, r1c, idegcc, r0t, r1t, idegct, *rest):
    slotmap = rest[0]
    nsb = 5 + ngr
    sides = (
        (b0c, b1c, invdc, ctrc, r0c, r1c, idegcc, rest[1:1 + nsb]),
        (b0t, b1t, invdt, ctrt, r0t, r1t, idegct, rest[1 + nsb:1 + 2 * nsb]),
    )
    esem, gsem = rest[1 + 2 * nsb:3 + 2 * nsb]
    c = lax.axis_index("c")
    s = lax.axis_index("s")
    base = (c * NS + s) * bpw
    iota16 = lax.broadcasted_iota(jnp.int32, (16,), 0)
    trash16 = jnp.full((16,), TRASH, jnp.int32)

    for (_, _, invdeg, center, _, _, _, sb) in sides:
      cbuf, _, _, degv, _ = sb[:5]
      pltpu.make_async_copy(center, cbuf.at[0], esem).start()
      pltpu.make_async_copy(invdeg, degv, esem).start()

    for (b0, b1, invdeg, center, r0, r1, idegc, sb) in sides:
      cbuf, rows0, rows1, degv, degc = sb[:5]
      slb = sb[5:]

      def fill(t, carry):
        slotmap[pl.ds(t * 16, 16)] = trash16
        return carry
      lax.fori_loop(0, n_pad // 16, fill, 0)
      pltpu.make_async_copy(center, cbuf.at[0], esem).wait()

      def scat(g, carry):
        idx = cbuf[0, pl.ds(g * 16, 16)]
        plsc.store_scatter(slotmap, [idx], g * 16 + iota16)
        return carry
      lax.fori_loop(0, n_b // 16, scat, 0)

      pltpu.make_async_copy(invdeg, degv, esem).wait()
      for g in range(ngr):
        cidx = cbuf[0, pl.ds(base + g * 16, 16)]
        sl16 = plsc.load_gather(slotmap, [cidx])
        slb[g][0, pl.ds(0, 16)] = sl16
        pltpu.make_async_copy(b0.at[slb[g].at[0]],
                              rows0.at[pl.ds(g * 16, 16)], gsem).start()
        pltpu.make_async_copy(b1.at[slb[g].at[0]],
                              rows1.at[pl.ds(g * 16, 16)], gsem).start()
        degc[pl.ds(g * 16, 16)] = plsc.load_gather(degv, [cidx])

    for (b0, b1, _, _, r0, r1, idegc, sb) in sides:
      _, rows0, rows1, _, degc = sb[:5]
      slb = sb[5:]
      for g in range(ngr):
        pltpu.make_async_copy(b0.at[slb[g].at[0]],
                              rows0.at[pl.ds(g * 16, 16)], gsem).wait()
        pltpu.make_async_copy(b1.at[slb[g].at[0]],
                              rows1.at[pl.ds(g * 16, 16)], gsem).wait()
      pltpu.sync_copy(rows0, r0.at[pl.ds(base, bpw)])
      pltpu.sync_copy(rows1, r1.at[pl.ds(base, bpw)])
      pltpu.sync_copy(degc, idegc.at[pl.ds(base, bpw)])

  return pl.kernel(body, out_type=out_type, mesh=mesh, scratch_types=scratch,
                   compiler_params=_SC_PARAMS)


def _conv_dense(n_pad, d_in, d_out):
  blk = 1280
  grid = (n_pad // blk,)

  def body(a0, a1, degp, w, bb, h, invd):
    deg = jnp.sum(degp[...], axis=1)
    inv = 1.0 / jnp.maximum(deg, 1.0)
    agg = (a0[...] + a1[...]) * inv[:, None]
    h[...] = jnp.maximum(
        jnp.dot(agg, w[...], preferred_element_type=jnp.float32) + bb[...], 0.0)
    invd[...] = inv[:, None]

  return pl.pallas_call(
      body,
      grid=grid,
      in_specs=[
          pl.BlockSpec((blk, d_in), lambda i: (i, 0)),
          pl.BlockSpec((blk, d_in), lambda i: (i, 0)),
          pl.BlockSpec((blk, NW), lambda i: (i, 0)),
          pl.BlockSpec((d_in, d_out), lambda i: (0, 0)),
          pl.BlockSpec((1, d_out), lambda i: (0, 0)),
      ],
      out_specs=[
          pl.BlockSpec((blk, d_out), lambda i: (i, 0)),
          pl.BlockSpec((blk, 1), lambda i: (i, 0)),
      ],
      out_shape=[
          jax.ShapeDtypeStruct((n_pad, d_out), jnp.float32),
          jax.ShapeDtypeStruct((n_pad, 1), jnp.float32),
      ],
  )


def _final_kernel(h, n_b, h2):
  def body(r0c, r1c, idc, r0t, r1t, idt, cw2, cb2, tw2, tb2,
           pw1, pb1, pw2, pb2, zp, zt):
    zc = jnp.dot((r0c[...] + r1c[...]) * idc[...], cw2[...],
                 preferred_element_type=jnp.float32) + cb2[...]
    hid = jnp.maximum(
        jnp.dot(zc, pw1[...], preferred_element_type=jnp.float32) + pb1[...],
        0.0)
    zp[...] = jnp.dot(hid, pw2[...],
                      preferred_element_type=jnp.float32) + pb2[...]
    zt[...] = jnp.dot((r0t[...] + r1t[...]) * idt[...], tw2[...],
                      preferred_element_type=jnp.float32) + tb2[...]

  return pl.pallas_call(
      body,
      out_shape=[jax.ShapeDtypeStruct((n_b, h), jnp.float32),
                 jax.ShapeDtypeStruct((n_b, h), jnp.float32)])


def kernel(x_context, edge_index_context, center_mask_context,
           x_target, edge_index_target, center_mask_target,
           cW1, cb1, cW2, cb2, tW1, tb1, tW2, tb2,
           pW1, pb1, pW2, pb2):
  n, d = x_context.shape
  e = edge_index_context.shape[1]
  nb = center_mask_context.shape[0]
  h = cW1.shape[1]
  h2 = pW1.shape[1]
  n_pad = -(-n // (NS * 128)) * (NS * 128)

  seg_deg = _seg_sum_kernel(n_pad, d, e, True)
  seg2 = _conv2_filtered_kernel(n_pad, h, e, nb)
  conv = _conv_dense(n_pad, d, h)
  gath = _center_gather_kernel(n_pad, h, nb)
  fin = _final_kernel(h, nb, h2)

  def enc(x, ei, ctr, w1, bias1):
    src, dst = ei[0], ei[1]
    a0, a1, degp = seg_deg(x, src, dst)
    hdn, invd = conv(a0, a1, degp.T, w1, bias1.reshape(1, h))
    bout = seg2(hdn, src, dst, ctr)
    return bout, invd.reshape(-1)

  boutc, invdc = enc(x_context, edge_index_context, center_mask_context,
                     cW1, cb1)
  boutt, invdt = enc(x_target, edge_index_target, center_mask_target,
                     tW1, tb1)
  r0c, r1c, idc, r0t, r1t, idt = gath(
      boutc[0], boutc[1], invdc, center_mask_context,
      boutt[0], boutt[1], invdt, center_mask_target)
  zp, zt = fin(r0c, r1c, idc.reshape(nb, 1), r0t, r1t, idt.reshape(nb, 1),
               cW2, cb2.reshape(1, h), tW2, tb2.reshape(1, h),
               pW1, pb1.reshape(1, h2), pW2, pb2.reshape(1, h))
  return (zp, zt)


# speedup vs baseline: 15.7426x; 1.0095x over previous
import jax
import jax.numpy as jnp
from jax import lax
from jax.experimental import pallas as pl
from jax.experimental.pallas import tpu as pltpu
from jax.experimental.pallas import tpu_sc as plsc

NC = 2
NS = 16
NW = NC * NS

_SC_PARAMS = pltpu.CompilerParams(needs_layout_passes=False)


_K = 40
_G = 5


def _seg_sum_kernel(n_pad, feat, n_edges, with_deg):
  K, G = _K, _G
  epw = n_edges // NW
  nchunk = epw // K
  ng = nchunk // G
  nouter = ng // 2
  assert epw * NW == n_edges and K * nchunk == epw and G * ng == nchunk
  assert ng % 2 == 0 and n_pad % (NS * 128) == 0
  rps = n_pad // NS
  mesh = plsc.VectorSubcoreMesh(core_axis_name="c", subcore_axis_name="s")
  out_type = [jax.ShapeDtypeStruct((n_pad, feat), jnp.float32),
              jax.ShapeDtypeStruct((n_pad, feat), jnp.float32)]
  if with_deg:
    out_type.append(jax.ShapeDtypeStruct((NW, n_pad), jnp.float32))
  scratch = [pltpu.VMEM((1, K), jnp.int32)] * (2 * G) \
    + [pltpu.VMEM((1, K), jnp.int32)] * (2 * G) \
    + [pltpu.VMEM((1, K), jnp.int32)] * G \
    + [
      pltpu.VMEM((G, K, feat), jnp.float32),
      pltpu.VMEM((64, feat), jnp.float32),
      pltpu.VMEM_SHARED((n_pad, feat), jnp.float32),
      pltpu.SemaphoreType.DMA,
      pltpu.SemaphoreType.DMA,
  ] + [pltpu.SemaphoreType.DMA] * G \
    + [pltpu.SemaphoreType.DMA] * G \
    + [pltpu.SemaphoreType.DMA]
  if with_deg:
    scratch.append(pltpu.VMEM((n_pad,), jnp.float32))

  def body(table, src_r, dst_r, acc0, acc1, *rest):
    if with_deg:
      degp = rest[0]
      rest = rest[1:]
    srcb = rest[:2 * G]
    dstb = rest[2 * G:4 * G]
    sdst = rest[4 * G:5 * G]
    rows, zbuf, acc_sh = rest[5 * G:5 * G + 3]
    off0 = 5 * G + 3
    isem = rest[off0:off0 + 2]
    gsem = rest[off0 + 2:off0 + 2 + G]
    ssem = rest[off0 + 2 + G:off0 + 2 + 2 * G]
    zsem = rest[off0 + 2 + 2 * G]
    degbuf = rest[off0 + 3 + 2 * G] if with_deg else None
    c = lax.axis_index("c")
    s = lax.axis_index("s")
    wid = c * NS + s
    wbase = wid * epw
    z16 = jnp.zeros((16,), jnp.float32)
    ones = jnp.ones((16,), jnp.float32)
    tailmask = lax.broadcasted_iota(jnp.int32, (16,), 0) >= 8
    gpr = feat // 16

    def zb(t, carry):
      zbuf[t // gpr, pl.ds((t % gpr) * 16, 16)] = z16
      return carry
    lax.fori_loop(0, 64 * gpr, zb, 0)
    for r in range(rps // 64):
      pltpu.make_async_copy(
          zbuf, acc_sh.at[pl.ds(s * rps + r * 64, 64)], zsem).start()
    if with_deg:
      def zd(t, carry):
        degbuf[pl.ds(t * 16, 16)] = z16
        return carry
      lax.fori_loop(0, n_pad // 16, zd, 0)
    for r in range(rps // 64):
      pltpu.make_async_copy(
          zbuf, acc_sh.at[pl.ds(s * rps + r * 64, 64)], zsem).wait()
    plsc.subcore_barrier()

    def fire_idx_group(gi, q):
      off = pl.multiple_of(wbase + gi * (G * K), 8)
      for uu in range(G):
        pltpu.make_async_copy(src_r.at[pl.ds(off + uu * K, K)],
                              srcb[q * G + uu].at[0], isem[q]).start()
        pltpu.make_async_copy(dst_r.at[pl.ds(off + uu * K, K)],
                              dstb[q * G + uu].at[0], isem[q]).start()

    def wait_idx_group(q):
      off = pl.multiple_of(wbase, 8)
      for uu in range(G):
        pltpu.make_async_copy(src_r.at[pl.ds(off + uu * K, K)],
                              srcb[q * G + uu].at[0], isem[q]).wait()
        pltpu.make_async_copy(dst_r.at[pl.ds(off + uu * K, K)],
                              dstb[q * G + uu].at[0], isem[q]).wait()

    def fire_gather(q, u):
      pltpu.make_async_copy(table.at[srcb[q * G + u].at[0]],
                            rows.at[u], gsem[u]).start()

    def wait_gather(u):
      pltpu.make_async_copy(table.at[srcb[u].at[0]],
                            rows.at[u], gsem[u]).wait()

    def fire_scatter(u):
      pltpu.async_copy(rows.at[u], acc_sh.at[sdst[u].at[0]],
                       ssem[u], add=True)

    def wait_scatter(u):
      pltpu.make_async_copy(rows.at[u], acc_sh.at[sdst[u].at[0]],
                            ssem[u]).wait()

    def drain_chunk(q, u):
      wait_gather(u)
      d = dstb[q * G + u]
      for o in (0, 16, 24):
        sdst[u][0, pl.ds(o, 16)] = d[0, pl.ds(o, 16)]
      fire_scatter(u)
      if with_deg:
        d = dstb[q * G + u]
        plsc.addupdate_scatter(degbuf, [d[0, pl.ds(0, 16)]], ones)
        plsc.addupdate_scatter(degbuf, [d[0, pl.ds(16, 16)]], ones)
        plsc.addupdate_scatter(degbuf, [d[0, pl.ds(24, 16)]], ones,
                               mask=tailmask)

    offp = pl.multiple_of(wbase, 8)
    for uu in range(G):
      pltpu.sync_copy(src_r.at[pl.ds(offp + uu * K, K)], srcb[uu].at[0])
      pltpu.sync_copy(dst_r.at[pl.ds(offp + uu * K, K)], dstb[uu].at[0])
    fire_idx_group(1, 1)
    fire_gather(0, 0)
    fire_gather(0, 1)
    fire_gather(0, 2)
    fire_gather(0, 3)

    def outer(gg, carry):
      for pp in (0, 1):
        for u in range(G):
          if u < 1:
            w = u + 4
            if pp == 0:
              @pl.when(gg > 0)
              def _(w=w):
                wait_scatter(w)
            else:
              wait_scatter(w)
            fire_gather(pp, w)
          else:
            w = u - 1
            if pp == 0:
              if u == 1:
                wait_idx_group(1)
              wait_scatter(w)
              fire_gather(1, w)
            else:
              @pl.when(gg < nouter - 1)
              def _(w=w, u=u):
                if u == 1:
                  wait_idx_group(0)
                wait_scatter(w)
                fire_gather(0, w)
          drain_chunk(pp, u)
          if u == 4:
            @pl.when(gg < nouter - 1)
            def _(pp=pp):
              fire_idx_group(gg * 2 + 2 + pp, pp)
      return carry

    lax.fori_loop(0, nouter, outer, 0)
    for u in range(G):
      wait_scatter(u)
    plsc.subcore_barrier()

    @pl.when(c == 0)
    def _():
      for r in range(rps // 128):
        sl = pl.ds(s * rps + r * 128, 128)
        pltpu.make_async_copy(acc_sh.at[sl], acc0.at[sl], zsem).start()
      for r in range(rps // 128):
        sl = pl.ds(s * rps + r * 128, 128)
        pltpu.make_async_copy(acc_sh.at[sl], acc0.at[sl], zsem).wait()

    @pl.when(c == 1)
    def _():
      for r in range(rps // 128):
        sl = pl.ds(s * rps + r * 128, 128)
        pltpu.make_async_copy(acc_sh.at[sl], acc1.at[sl], zsem).start()
      for r in range(rps // 128):
        sl = pl.ds(s * rps + r * 128, 128)
        pltpu.make_async_copy(acc_sh.at[sl], acc1.at[sl], zsem).wait()

    if with_deg:
      pltpu.sync_copy(degbuf, degp.at[wid])

  return pl.kernel(body, out_type=tuple(out_type), mesh=mesh,
                   scratch_types=tuple(scratch), compiler_params=_SC_PARAMS)


def _conv2_filtered_kernel(n_pad, feat, n_edges, n_b):
  epw = n_edges // NW
  SP = 2 * n_b
  TRASH = n_b
  R = 5
  assert epw % 16 == 0 and n_b % 128 == 0
  rps = SP // NS
  dps = n_b // NS
  mesh = plsc.VectorSubcoreMesh(core_axis_name="c", subcore_axis_name="s")
  out_type = jax.ShapeDtypeStruct((NC, n_b, feat), jnp.float32)
  scratch = (
      pltpu.VMEM((epw,), jnp.int32),
      pltpu.VMEM((epw,), jnp.int32),
      pltpu.VMEM((n_pad,), jnp.int32),
      pltpu.VMEM((epw + 16,), jnp.int32),
      pltpu.VMEM((epw + 16,), jnp.int32),
      pltpu.VMEM((1, n_b), jnp.int32),
      pltpu.VMEM((R, 16, feat), jnp.float32),
      pltpu.VMEM((64, feat), jnp.float32),
      pltpu.VMEM_SHARED((SP, feat), jnp.float32),
      pltpu.SemaphoreType.DMA,
  ) + (pltpu.SemaphoreType.DMA,) * R \
    + (pltpu.SemaphoreType.DMA,) * R \
    + (pltpu.SemaphoreType.DMA,)

  def body(table, src_r, dst_r, center, bout,
           srcv, dstv, slotmap, fsrc, fslot, cbuf, rows, zbuf, acc_sh,
           esem, *sems):
    gsem = sems[:R]
    ssem = sems[R:2 * R]
    zsem = sems[2 * R]
    c = lax.axis_index("c")
    s = lax.axis_index("s")
    wid = c * NS + s
    wbase = pl.multiple_of(wid * epw, 8)
    z16 = jnp.zeros((16,), jnp.float32)
    zi16 = jnp.zeros((16,), jnp.int32)
    iota16 = lax.broadcasted_iota(jnp.int32, (16,), 0)
    gpr = feat // 16

    pltpu.make_async_copy(src_r.at[pl.ds(wbase, epw)], srcv, esem).start()
    pltpu.make_async_copy(dst_r.at[pl.ds(wbase, epw)], dstv, esem).start()
    pltpu.make_async_copy(center, cbuf.at[0], esem).start()

    def zb(t, carry):
      zbuf[t // gpr, pl.ds((t % gpr) * 16, 16)] = z16
      return carry
    lax.fori_loop(0, 64 * gpr, zb, 0)
    for r in range(rps // 64):
      pltpu.make_async_copy(
          zbuf, acc_sh.at[pl.ds(s * rps + r * 64, 64)], zsem).start()

    trash16 = jnp.full((16,), TRASH, jnp.int32)

    def fill(t, carry):
      slotmap[pl.ds(t * 16, 16)] = trash16
      return carry
    lax.fori_loop(0, n_pad // 16, fill, 0)
    pltpu.make_async_copy(center, cbuf.at[0], esem).wait()

    def scat(g, carry):
      idx = cbuf[0, pl.ds(g * 16, 16)]
      plsc.store_scatter(slotmap, [idx], g * 16 + iota16)
      return carry
    lax.fori_loop(0, n_b // 16, scat, 0)

    pltpu.make_async_copy(src_r.at[pl.ds(wbase, epw)], srcv, esem).wait()
    pltpu.make_async_copy(dst_r.at[pl.ds(wbase, epw)], dstv, esem).wait()
    for r in range(rps // 64):
      pltpu.make_async_copy(
          zbuf, acc_sh.at[pl.ds(s * rps + r * 64, 64)], zsem).wait()
    plsc.subcore_barrier()

    def filt(i, ptr):
      d16 = dstv[pl.ds(i * 16, 16)]
      s16 = srcv[pl.ds(i * 16, 16)]
      sl16 = plsc.load_gather(slotmap, [d16])
      m = sl16 < TRASH
      plsc.store_compressed(fsrc.at[pl.ds(ptr, 16)], s16, mask=m)
      plsc.store_compressed(fslot.at[pl.ds(ptr, 16)], sl16, mask=m)
      cnt = plsc.all_reduce_population_count(m)[0]
      return ptr + cnt
    nf = lax.fori_loop(0, epw // 16, filt, jnp.int32(0))
    fsrc[pl.ds(nf, 16)] = zi16
    fslot[pl.ds(nf, 16)] = trash16
    t2 = (nf + 15) // 16

    def fire_gather(j, u):
      sidx = fsrc[pl.ds(j * 16, 16)]
      pltpu.make_async_copy(table.at[sidx], rows.at[u], gsem[u]).start()

    def wait_gather(u):
      pltpu.make_async_copy(table.at[zi16], rows.at[u], gsem[u]).wait()

    def fire_scatter(j, u):
      didx = fslot[pl.ds(j * 16, 16)]
      pltpu.async_copy(rows.at[u], acc_sh.at[didx], ssem[u], add=True)

    def wait_scatter(u):
      pltpu.make_async_copy(rows.at[u], acc_sh.at[zi16], ssem[u]).wait()

    for jp in range(3):
      @pl.when(jp < t2)
      def _(jp=jp):
        fire_gather(jp, jp)

    def pb(t, carry):
      for u in range(R):
        j = t * R + u
        j2 = j + 3
        w = (u + 3) % R

        @pl.when(jnp.logical_and(j2 < t2, j2 >= R))
        def _(w=w):
          wait_scatter(w)

        @pl.when(j2 < t2)
        def _(j2=j2, w=w):
          fire_gather(j2, w)

        @pl.when(j < t2)
        def _(j=j, u=u):
          wait_gather(u)
          fire_scatter(j, u)
      return carry
    lax.fori_loop(0, (t2 + R - 1) // R, pb, 0)
    for u in range(R):
      @pl.when(u < t2)
      def _(u=u):
        wait_scatter(u)
    plsc.subcore_barrier()

    pltpu.sync_copy(acc_sh.at[pl.ds(s * dps, dps)],
                    bout.at[c].at[pl.ds(s * dps, dps)])

  return pl.kernel(body, out_type=out_type, mesh=mesh, scratch_types=scratch,
                   compiler_params=_SC_PARAMS)


def _center_gather_kernel(n_pad, feat, n_b):
  bpw = n_b // NW
  TRASH = n_b
  assert bpw * NW == n_b and bpw % 16 == 0
  ngr = bpw // 16
  mesh = plsc.VectorSubcoreMesh(core_axis_name="c", subcore_axis_name="s")
  side_out = (jax.ShapeDtypeStruct((n_b, feat), jnp.float32),
              jax.ShapeDtypeStruct((n_b, feat), jnp.float32),
              jax.ShapeDtypeStruct((n_b,), jnp.float32))
  out_type = side_out + side_out
  side_scratch = (
      pltpu.VMEM((1, n_b), jnp.int32),
      pltpu.VMEM((bpw, feat), jnp.float32),
      pltpu.VMEM((bpw, feat), jnp.float32),
      pltpu.VMEM((n_pad,), jnp.float32),
      pltpu.VMEM((bpw,), jnp.float32),
  ) + tuple(pltpu.VMEM((1, 16), jnp.int32) for _ in range(ngr))
  scratch = (pltpu.VMEM((n_pad,), jnp.int32),) + side_scratch + side_scratch \
      + (pltpu.SemaphoreType.DMA, pltpu.SemaphoreType.DMA)

  def body(b0c, b1c, invdc, ctrc, b0t, b1t, invdt, ctrt,
           r0c, r1c, idegcc, r0t, r1t, idegct, *rest):
    slotmap = rest[0]
    nsb = 5 + ngr
    sides = (
        (b0c, b1c, invdc, ctrc, r0c, r1c, idegcc, rest[1:1 + nsb]),
        (b0t, b1t, invdt, ctrt, r0t, r1t, idegct, rest[1 + nsb:1 + 2 * nsb]),
    )
    esem, gsem = rest[1 + 2 * nsb:3 + 2 * nsb]
    c = lax.axis_index("c")
    s = lax.axis_index("s")
    base = (c * NS + s) * bpw
    iota16 = lax.broadcasted_iota(jnp.int32, (16,), 0)
    trash16 = jnp.full((16,), TRASH, jnp.int32)

    for (_, _, invdeg, center, _, _, _, sb) in sides:
      cbuf, _, _, degv, _ = sb[:5]
      pltpu.make_async_copy(center, cbuf.at[0], esem).start()
      pltpu.make_async_copy(invdeg, degv, esem).start()

    for (b0, b1, invdeg, center, r0, r1, idegc, sb) in sides:
      cbuf, rows0, rows1, degv, degc = sb[:5]
      slb = sb[5:]

      def fill(t, carry):
        slotmap[pl.ds(t * 16, 16)] = trash16
        return carry
      lax.fori_loop(0, n_pad // 16, fill, 0)
      pltpu.make_async_copy(center, cbuf.at[0], esem).wait()

      def scat(g, carry):
        idx = cbuf[0, pl.ds(g * 16, 16)]
        plsc.store_scatter(slotmap, [idx], g * 16 + iota16)
        return carry
      lax.fori_loop(0, n_b // 16, scat, 0)

      pltpu.make_async_copy(invdeg, degv, esem).wait()
      for g in range(ngr):
        cidx = cbuf[0, pl.ds(base + g * 16, 16)]
        sl16 = plsc.load_gather(slotmap, [cidx])
        slb[g][0, pl.ds(0, 16)] = sl16
        pltpu.make_async_copy(b0.at[slb[g].at[0]],
                              rows0.at[pl.ds(g * 16, 16)], gsem).start()
        pltpu.make_async_copy(b1.at[slb[g].at[0]],
                              rows1.at[pl.ds(g * 16, 16)], gsem).start()
        degc[pl.ds(g * 16, 16)] = plsc.load_gather(degv, [cidx])

    for (b0, b1, _, _, r0, r1, idegc, sb) in sides:
      _, rows0, rows1, _, degc = sb[:5]
      slb = sb[5:]
      for g in range(ngr):
        pltpu.make_async_copy(b0.at[slb[g].at[0]],
                              rows0.at[pl.ds(g * 16, 16)], gsem).wait()
        pltpu.make_async_copy(b1.at[slb[g].at[0]],
                              rows1.at[pl.ds(g * 16, 16)], gsem).wait()
      pltpu.sync_copy(rows0, r0.at[pl.ds(base, bpw)])
      pltpu.sync_copy(rows1, r1.at[pl.ds(base, bpw)])
      pltpu.sync_copy(degc, idegc.at[pl.ds(base, bpw)])

  return pl.kernel(body, out_type=out_type, mesh=mesh, scratch_types=scratch,
                   compiler_params=_SC_PARAMS)


def _conv_dense(n_pad, d_in, d_out):
  blk = 1280
  grid = (n_pad // blk,)

  def body(a0, a1, degp, w, bb, h, invd):
    deg = jnp.sum(degp[...], axis=1)
    inv = 1.0 / jnp.maximum(deg, 1.0)
    agg = (a0[...] + a1[...]) * inv[:, None]
    h[...] = jnp.maximum(
        jnp.dot(agg, w[...], preferred_element_type=jnp.float32) + bb[...], 0.0)
    invd[...] = inv[:, None]

  return pl.pallas_call(
      body,
      grid=grid,
      in_specs=[
          pl.BlockSpec((blk, d_in), lambda i: (i, 0)),
          pl.BlockSpec((blk, d_in), lambda i: (i, 0)),
          pl.BlockSpec((blk, NW), lambda i: (i, 0)),
          pl.BlockSpec((d_in, d_out), lambda i: (0, 0)),
          pl.BlockSpec((1, d_out), lambda i: (0, 0)),
      ],
      out_specs=[
          pl.BlockSpec((blk, d_out), lambda i: (i, 0)),
          pl.BlockSpec((blk, 1), lambda i: (i, 0)),
      ],
      out_shape=[
          jax.ShapeDtypeStruct((n_pad, d_out), jnp.float32),
          jax.ShapeDtypeStruct((n_pad, 1), jnp.float32),
      ],
  )


def _final_kernel(h, n_b, h2):
  def body(r0c, r1c, idc, r0t, r1t, idt, cw2, cb2, tw2, tb2,
           pw1, pb1, pw2, pb2, zp, zt):
    zc = jnp.dot((r0c[...] + r1c[...]) * idc[...], cw2[...],
                 preferred_element_type=jnp.float32) + cb2[...]
    hid = jnp.maximum(
        jnp.dot(zc, pw1[...], preferred_element_type=jnp.float32) + pb1[...],
        0.0)
    zp[...] = jnp.dot(hid, pw2[...],
                      preferred_element_type=jnp.float32) + pb2[...]
    zt[...] = jnp.dot((r0t[...] + r1t[...]) * idt[...], tw2[...],
                      preferred_element_type=jnp.float32) + tb2[...]

  return pl.pallas_call(
      body,
      out_shape=[jax.ShapeDtypeStruct((n_b, h), jnp.float32),
                 jax.ShapeDtypeStruct((n_b, h), jnp.float32)])


def kernel(x_context, edge_index_context, center_mask_context,
           x_target, edge_index_target, center_mask_target,
           cW1, cb1, cW2, cb2, tW1, tb1, tW2, tb2,
           pW1, pb1, pW2, pb2):
  n, d = x_context.shape
  e = edge_index_context.shape[1]
  nb = center_mask_context.shape[0]
  h = cW1.shape[1]
  h2 = pW1.shape[1]
  n_pad = -(-n // (NS * 128)) * (NS * 128)

  seg_deg = _seg_sum_kernel(n_pad, d, e, True)
  seg2 = _conv2_filtered_kernel(n_pad, h, e, nb)
  conv = _conv_dense(n_pad, d, h)
  gath = _center_gather_kernel(n_pad, h, nb)
  fin = _final_kernel(h, nb, h2)

  def enc(x, ei, ctr, w1, bias1):
    src, dst = ei[0], ei[1]
    a0, a1, degp = seg_deg(x, src, dst)
    hdn, invd = conv(a0, a1, degp.T, w1, bias1.reshape(1, h))
    bout = seg2(hdn, src, dst, ctr)
    return bout, invd.reshape(-1)

  boutc, invdc = enc(x_context, edge_index_context, center_mask_context,
                     cW1, cb1)
  boutt, invdt = enc(x_target, edge_index_target, center_mask_target,
                     tW1, tb1)
  r0c, r1c, idc, r0t, r1t, idt = gath(
      boutc[0], boutc[1], invdc, center_mask_context,
      boutt[0], boutt[1], invdt, center_mask_target)
  zp, zt = fin(r0c, r1c, idc.reshape(nb, 1), r0t, r1t, idt.reshape(nb, 1),
               cW2, cb2.reshape(1, h), tW2, tb2.reshape(1, h),
               pW1, pb1.reshape(1, h2), pW2, pb2.reshape(1, h))
  return (zp, zt)
